# baseline (speedup 1.0000x reference)
import functools

import jax
import jax.numpy as jnp
from jax.experimental import pallas as pl

NUM_GRAPHS = 512
EDGE_BLK = 8000


def _ssp(x):
    return jnp.maximum(x, 0.0) + jnp.log1p(jnp.exp(-jnp.abs(x))) - 0.6931471805599453


def _dense(x, W, b):
    return jax.lax.dot_general(x, W, (((1,), (1,)), ((), ())),
                               preferred_element_type=jnp.float32) + b


def _edge_kernel(xs_ref, xd_ref, ea_ref, ub_ref,
                 w1_ref, b1_ref, w2_ref, b2_ref,
                 v1_ref, c1_ref, v2_ref, c2_ref, v3_ref, c3_ref,
                 enew_ref, eout_ref, *, skip_is_ep):
    ea = ea_ref[...]
    e_p = _ssp(_dense(ea, w1_ref[...], b1_ref[...]))
    e_p = _ssp(_dense(e_p, w2_ref[...], b2_ref[...]))
    h = jnp.concatenate([xs_ref[...], xd_ref[...], e_p, ub_ref[...]], axis=1)
    h = _ssp(_dense(h, v1_ref[...], c1_ref[...]))
    h = _ssp(_dense(h, v2_ref[...], c2_ref[...]))
    e_new = _ssp(_dense(h, v3_ref[...], c3_ref[...]))
    enew_ref[...] = e_new
    if skip_is_ep:
        eout_ref[...] = e_new + e_p
    else:
        eout_ref[...] = e_new + ea


def _edge_pipeline(p, xs, xd, edge_attr, ub, skip_is_ep):
    E = xs.shape[0]
    n_blk = E // EDGE_BLK
    w1, b1 = p['pre_e'][0]['W'], p['pre_e'][0]['b']
    w2, b2 = p['pre_e'][1]['W'], p['pre_e'][1]['b']
    v1, c1 = p['phi_e'][0]['W'], p['phi_e'][0]['b']
    v2, c2 = p['phi_e'][1]['W'], p['phi_e'][1]['b']
    v3, c3 = p['phi_e'][2]['W'], p['phi_e'][2]['b']

    def blk(feat):
        return pl.BlockSpec((EDGE_BLK, feat), lambda n: (n, 0))

    def const(shape):
        return pl.BlockSpec(shape, lambda n: tuple(0 for _ in shape))

    e_in_dim = edge_attr.shape[1]
    e_new, e_out = pl.pallas_call(
        functools.partial(_edge_kernel, skip_is_ep=skip_is_ep),
        grid=(n_blk,),
        in_specs=[
            blk(32), blk(32), blk(e_in_dim), blk(32),
            const(w1.shape), const((1, 64)), const(w2.shape), const((1, 32)),
            const(v1.shape), const((1, 64)), const(v2.shape), const((1, 64)),
            const(v3.shape), const((1, 32)),
        ],
        out_specs=[blk(32), blk(32)],
        out_shape=[jax.ShapeDtypeStruct((E, 32), jnp.float32),
                   jax.ShapeDtypeStruct((E, 32), jnp.float32)],
    )(xs, xd, edge_attr, ub,
      w1, b1.reshape(1, -1), w2, b2.reshape(1, -1),
      v1, c1.reshape(1, -1), v2, c2.reshape(1, -1), v3, c3.reshape(1, -1))
    return e_new, e_out


def _mlp(ps, x):
    for q in ps:
        x = _ssp(x @ q['W'].T + q['b'])
    return x


def _scatter_mean(vals, idx, n):
    s = jax.ops.segment_sum(vals, idx, num_segments=n)
    cnt = jax.ops.segment_sum(jnp.ones((vals.shape[0],), vals.dtype), idx,
                              num_segments=n)
    return s / jnp.clip(cnt, 1.0)[:, None]


def kernel(x, edge_index, edge_attr, state, batch, bond_batch, params):
    p_all = params
    xv = p_all['emb'][x]
    src, dst = edge_index[0], edge_index[1]
    n_nodes = xv.shape[0]

    def module(p, xv, edge_attr, state, inner_skip):
        x_in, u_in = xv, state
        x_p = _mlp(p['pre_v'], xv)
        u_p = _mlp(p['pre_u'], state)
        if inner_skip:
            x_in, u_in = x_p, u_p
        xs = x_p[src]
        xd = x_p[dst]
        ub = u_p[bond_batch]
        e_new, e_out = _edge_pipeline(p, xs, xd, edge_attr, ub, inner_skip)
        agg = _scatter_mean(e_new, dst, n_nodes)
        v_feat = jnp.concatenate([agg, x_p, u_p[batch]], axis=1)
        v_new = _mlp(p['phi_v'], v_feat)
        u_e = _scatter_mean(e_new, bond_batch, NUM_GRAPHS)
        u_v = _scatter_mean(v_new, batch, NUM_GRAPHS)
        u_feat = jnp.concatenate([u_e, u_v, u_p], axis=1)
        u_new = _mlp(p['phi_u'], u_feat)
        return v_new + x_in, e_out, u_new + u_in

    xv, edge_attr, state = module(p_all['m1'], xv, edge_attr, state, True)
    xv, edge_attr, state = module(p_all['m2'], xv, edge_attr, state, False)
    xv, edge_attr, state = module(p_all['m3'], xv, edge_attr, state, False)

    def set2set(p, xx, bb, num_graphs):
        in_ch = xx.shape[1]
        h = jnp.zeros((num_graphs, in_ch), xx.dtype)
        c = jnp.zeros((num_graphs, in_ch), xx.dtype)
        q_star = jnp.zeros((num_graphs, 2 * in_ch), xx.dtype)
        gates = q_star @ p['W_ih'].T + h @ p['W_hh'].T + p['b_ih'] + p['b_hh']
        i, f, g, o = jnp.split(gates, 4, axis=1)
        c = jax.nn.sigmoid(f) * c + jax.nn.sigmoid(i) * jnp.tanh(g)
        q = jax.nn.sigmoid(o) * jnp.tanh(c)
        e = jnp.sum(xx * q[bb], axis=1)
        m = jax.ops.segment_max(e, bb, num_segments=num_graphs)
        m = jnp.where(jnp.isfinite(m), m, 0.0)
        a = jnp.exp(e - m[bb])
        denom = jax.ops.segment_sum(a, bb, num_segments=num_graphs)
        a = a / (denom[bb] + 1e-16)
        r = jax.ops.segment_sum(a[:, None] * xx, bb, num_segments=num_graphs)
        return jnp.concatenate([q, r], axis=1)

    xg = set2set(p_all['sv'], xv, batch, NUM_GRAPHS)
    eg = set2set(p_all['se'], edge_attr, bond_batch, NUM_GRAPHS)
    tmp = jnp.concatenate([xg, eg, state], axis=1)
    h = _ssp(tmp @ p_all['h0']['W'].T + p_all['h0']['b'])
    h = _ssp(h @ p_all['h1']['W'].T + p_all['h1']['b'])
    return h @ p_all['h2']['W'].T + p_all['h2']['b']


# speedup vs baseline: 2.1076x; 2.1076x over previous
import functools

import jax
import jax.numpy as jnp
from jax.experimental import pallas as pl

NUM_GRAPHS = 512
EDGE_BLK = 2000
NODE_BLK = 2000
_LOG2 = 0.6931471805599453


def _ssp(x):
    return jnp.maximum(x, 0.0) + jnp.log1p(jnp.exp(-jnp.abs(x))) - _LOG2


def _dg(x, w):
    return jax.lax.dot_general(x, w, (((1,), (1,)), ((), ())),
                               preferred_element_type=jnp.float32)


def _dgT(a, b):
    return jax.lax.dot_general(a, b, (((0,), (0,)), ((), ())),
                               preferred_element_type=jnp.float32)


def _matT(w, xT, b):
    return jax.lax.dot_general(w, xT, (((1,), (0,)), ((), ())),
                               preferred_element_type=jnp.float32) + b


def _const_spec(shape):
    return pl.BlockSpec(shape, lambda n: tuple(0 for _ in shape))


def _row_spec(blk, feat):
    return pl.BlockSpec((blk, feat), lambda n: (n, 0))


def _idx_spec(blk):
    return pl.BlockSpec((1, 1, blk), lambda n: (n, 0, 0))


def _edge_kernel(xs_ref, xd_ref, ea_ref, bb_ref, upT_ref, qe_ref,
                 w1_ref, b1_ref, w2_ref, b2_ref,
                 v1_ref, c1_ref, v2_ref, c2_ref, v3_ref, c3_ref,
                 enew_ref, eout_ref, ue_ref, bcnt_ref, me_ref,
                 *, skip_is_ep, compute_max):
    first = pl.program_id(0) == 0
    ea = ea_ref[...]
    e_p = _ssp(_dg(ea, w1_ref[...]) + b1_ref[...])
    e_p = _ssp(_dg(e_p, w2_ref[...]) + b2_ref[...])

    blk = ea.shape[0]
    bb = bb_ref[0, 0, :]
    onehot = (bb[:, None] == jax.lax.broadcasted_iota(
        jnp.int32, (blk, NUM_GRAPHS), 1)).astype(jnp.float32)
    ub = jax.lax.dot_general(onehot, upT_ref[...], (((1,), (1,)), ((), ())),
                             preferred_element_type=jnp.float32)

    h = jnp.concatenate([xs_ref[...], xd_ref[...], e_p, ub], axis=1)
    h = _ssp(_dg(h, v1_ref[...]) + c1_ref[...])
    h = _ssp(_dg(h, v2_ref[...]) + c2_ref[...])
    e_new = _ssp(_dg(h, v3_ref[...]) + c3_ref[...])
    enew_ref[...] = e_new
    if skip_is_ep:
        e_out = e_new + e_p
    else:
        e_out = e_new + ea
    eout_ref[...] = e_out

    @pl.when(first)
    def _():
        ue_ref[...] = jnp.zeros_like(ue_ref)
        bcnt_ref[...] = jnp.zeros_like(bcnt_ref)

    ue_ref[...] += _dgT(e_new, onehot)
    bcnt_ref[...] += jnp.sum(onehot, axis=0, keepdims=True)

    if compute_max:
        ee = jax.lax.dot_general(e_out, qe_ref[...], (((1,), (1,)), ((), ())),
                                 preferred_element_type=jnp.float32)
        masked = jnp.where(onehot > 0.5, ee, -jnp.inf)
        m_part = jnp.max(masked, axis=0, keepdims=True)

        @pl.when(first)
        def _():
            me_ref[...] = jnp.full_like(me_ref, -jnp.inf)

        me_ref[...] = jnp.maximum(me_ref[...], m_part)


def _edge_pipeline(p, xs, xd, edge_attr, bond3, upT, qe, skip_is_ep,
                   compute_max):
    E = xs.shape[0]
    n_blk = E // EDGE_BLK
    w1, b1 = p['pre_e'][0]['W'], p['pre_e'][0]['b']
    w2, b2 = p['pre_e'][1]['W'], p['pre_e'][1]['b']
    v1, c1 = p['phi_e'][0]['W'], p['phi_e'][0]['b']
    v2, c2 = p['phi_e'][1]['W'], p['phi_e'][1]['b']
    v3, c3 = p['phi_e'][2]['W'], p['phi_e'][2]['b']
    e_dim = edge_attr.shape[1]

    out = pl.pallas_call(
        functools.partial(_edge_kernel, skip_is_ep=skip_is_ep,
                          compute_max=compute_max),
        grid=(n_blk,),
        in_specs=[
            _row_spec(EDGE_BLK, 32), _row_spec(EDGE_BLK, 32),
            _row_spec(EDGE_BLK, e_dim), _idx_spec(EDGE_BLK),
            _const_spec((32, NUM_GRAPHS)), _const_spec((1, 32)),
            _const_spec(w1.shape), _const_spec((1, 64)),
            _const_spec(w2.shape), _const_spec((1, 32)),
            _const_spec(v1.shape), _const_spec((1, 64)),
            _const_spec(v2.shape), _const_spec((1, 64)),
            _const_spec(v3.shape), _const_spec((1, 32)),
        ],
        out_specs=[
            _row_spec(EDGE_BLK, 32), _row_spec(EDGE_BLK, 32),
            _const_spec((32, NUM_GRAPHS)), _const_spec((1, NUM_GRAPHS)),
            _const_spec((1, NUM_GRAPHS)),
        ],
        out_shape=[
            jax.ShapeDtypeStruct((E, 32), jnp.float32),
            jax.ShapeDtypeStruct((E, 32), jnp.float32),
            jax.ShapeDtypeStruct((32, NUM_GRAPHS), jnp.float32),
            jax.ShapeDtypeStruct((1, NUM_GRAPHS), jnp.float32),
            jax.ShapeDtypeStruct((1, NUM_GRAPHS), jnp.float32),
        ],
    )(xs, xd, edge_attr, bond3, upT, qe,
      w1, b1.reshape(1, -1), w2, b2.reshape(1, -1),
      v1, c1.reshape(1, -1), v2, c2.reshape(1, -1), v3, c3.reshape(1, -1))
    return out


def _node_pre_kernel(xv_ref, emb_ref, w1_ref, b1_ref, w2_ref, b2_ref,
                     xp_ref, *, with_emb):
    if with_emb:
        xi = xv_ref[0, 0, :]
        blk = xi.shape[0]
        oh = (xi[:, None] == jax.lax.broadcasted_iota(
            jnp.int32, (blk, 95), 1)).astype(jnp.float32)
        xv = jax.lax.dot_general(oh, emb_ref[...], (((1,), (0,)), ((), ())),
                                 preferred_element_type=jnp.float32)
    else:
        xv = xv_ref[...]
    h = _ssp(_dg(xv, w1_ref[...]) + b1_ref[...])
    xp_ref[...] = _ssp(_dg(h, w2_ref[...]) + b2_ref[...])


def _node_pre(p, xv_or_ids, emb, with_emb, n_nodes):
    n_blk = n_nodes // NODE_BLK
    w1, b1 = p['pre_v'][0]['W'], p['pre_v'][0]['b']
    w2, b2 = p['pre_v'][1]['W'], p['pre_v'][1]['b']
    if with_emb:
        first_spec = _idx_spec(NODE_BLK)
    else:
        first_spec = _row_spec(NODE_BLK, xv_or_ids.shape[1])
    return pl.pallas_call(
        functools.partial(_node_pre_kernel, with_emb=with_emb),
        grid=(n_blk,),
        in_specs=[first_spec, _const_spec(emb.shape),
                  _const_spec(w1.shape), _const_spec((1, 64)),
                  _const_spec(w2.shape), _const_spec((1, 32))],
        out_specs=_row_spec(NODE_BLK, 32),
        out_shape=jax.ShapeDtypeStruct((n_nodes, 32), jnp.float32),
    )(xv_or_ids, emb, w1, b1.reshape(1, -1), w2, b2.reshape(1, -1))


def _node_post_kernel(aggs_ref, deg_ref, xp_ref, xin_ref, b_ref, upT_ref,
                      qv_ref, w1_ref, b1_ref, w2_ref, b2_ref, w3_ref, b3_ref,
                      vout_ref, uv_ref, ncnt_ref, mv_ref, *, compute_max):
    first = pl.program_id(0) == 0
    agg = aggs_ref[...] / deg_ref[...]
    blk = agg.shape[0]
    bb = b_ref[0, 0, :]
    onehot = (bb[:, None] == jax.lax.broadcasted_iota(
        jnp.int32, (blk, NUM_GRAPHS), 1)).astype(jnp.float32)
    ub = jax.lax.dot_general(onehot, upT_ref[...], (((1,), (1,)), ((), ())),
                             preferred_element_type=jnp.float32)
    h = jnp.concatenate([agg, xp_ref[...], ub], axis=1)
    h = _ssp(_dg(h, w1_ref[...]) + b1_ref[...])
    h = _ssp(_dg(h, w2_ref[...]) + b2_ref[...])
    v_new = _ssp(_dg(h, w3_ref[...]) + b3_ref[...])
    v_out = v_new + xin_ref[...]
    vout_ref[...] = v_out

    @pl.when(first)
    def _():
        uv_ref[...] = jnp.zeros_like(uv_ref)
        ncnt_ref[...] = jnp.zeros_like(ncnt_ref)

    uv_ref[...] += _dgT(v_new, onehot)
    ncnt_ref[...] += jnp.sum(onehot, axis=0, keepdims=True)

    if compute_max:
        ev = jax.lax.dot_general(v_out, qv_ref[...], (((1,), (1,)), ((), ())),
                                 preferred_element_type=jnp.float32)
        masked = jnp.where(onehot > 0.5, ev, -jnp.inf)
        m_part = jnp.max(masked, axis=0, keepdims=True)

        @pl.when(first)
        def _():
            mv_ref[...] = jnp.full_like(mv_ref, -jnp.inf)

        mv_ref[...] = jnp.maximum(mv_ref[...], m_part)


def _node_post(p, agg_sum, degc, x_p, x_in, batch3, upT, qv, compute_max):
    n_nodes = x_p.shape[0]
    n_blk = n_nodes // NODE_BLK
    w1, b1 = p['phi_v'][0]['W'], p['phi_v'][0]['b']
    w2, b2 = p['phi_v'][1]['W'], p['phi_v'][1]['b']
    w3, b3 = p['phi_v'][2]['W'], p['phi_v'][2]['b']
    return pl.pallas_call(
        functools.partial(_node_post_kernel, compute_max=compute_max),
        grid=(n_blk,),
        in_specs=[
            _row_spec(NODE_BLK, 32), _row_spec(NODE_BLK, 1),
            _row_spec(NODE_BLK, 32), _row_spec(NODE_BLK, 32),
            _idx_spec(NODE_BLK), _const_spec((32, NUM_GRAPHS)),
            _const_spec((1, 32)),
            _const_spec(w1.shape), _const_spec((1, 64)),
            _const_spec(w2.shape), _const_spec((1, 64)),
            _const_spec(w3.shape), _const_spec((1, 32)),
        ],
        out_specs=[
            _row_spec(NODE_BLK, 32),
            _const_spec((32, NUM_GRAPHS)), _const_spec((1, NUM_GRAPHS)),
            _const_spec((1, NUM_GRAPHS)),
        ],
        out_shape=[
            jax.ShapeDtypeStruct((n_nodes, 32), jnp.float32),
            jax.ShapeDtypeStruct((32, NUM_GRAPHS), jnp.float32),
            jax.ShapeDtypeStruct((1, NUM_GRAPHS), jnp.float32),
            jax.ShapeDtypeStruct((1, NUM_GRAPHS), jnp.float32),
        ],
    )(agg_sum, degc, x_p, x_in, batch3, upT, qv,
      w1, b1.reshape(1, -1), w2, b2.reshape(1, -1), w3, b3.reshape(1, -1))


def _pre_u_kernel(sT_ref, w1_ref, b1_ref, w2_ref, b2_ref, o_ref):
    h = _ssp(_matT(w1_ref[...], sT_ref[...], b1_ref[...]))
    o_ref[...] = _ssp(_matT(w2_ref[...], h, b2_ref[...]))


def _pre_u(p, stateT):
    w1, b1 = p['pre_u'][0]['W'], p['pre_u'][0]['b']
    w2, b2 = p['pre_u'][1]['W'], p['pre_u'][1]['b']
    return pl.pallas_call(
        _pre_u_kernel,
        out_shape=jax.ShapeDtypeStruct((32, NUM_GRAPHS), jnp.float32),
    )(stateT, w1, b1.reshape(-1, 1), w2, b2.reshape(-1, 1))


def _phi_u_kernel(ue_ref, bcnt_ref, uv_ref, ncnt_ref, upT_ref, uinT_ref,
                  w1_ref, b1_ref, w2_ref, b2_ref, w3_ref, b3_ref, o_ref):
    u_e = ue_ref[...] / jnp.maximum(bcnt_ref[...], 1.0)
    u_v = uv_ref[...] / jnp.maximum(ncnt_ref[...], 1.0)
    h = jnp.concatenate([u_e, u_v, upT_ref[...]], axis=0)
    h = _ssp(_matT(w1_ref[...], h, b1_ref[...]))
    h = _ssp(_matT(w2_ref[...], h, b2_ref[...]))
    u_new = _ssp(_matT(w3_ref[...], h, b3_ref[...]))
    o_ref[...] = u_new + uinT_ref[...]


def _phi_u(p, ue_sumT, bcntT, uv_sumT, ncntT, upT, uinT):
    w1, b1 = p['phi_u'][0]['W'], p['phi_u'][0]['b']
    w2, b2 = p['phi_u'][1]['W'], p['phi_u'][1]['b']
    w3, b3 = p['phi_u'][2]['W'], p['phi_u'][2]['b']
    return pl.pallas_call(
        _phi_u_kernel,
        out_shape=jax.ShapeDtypeStruct((32, NUM_GRAPHS), jnp.float32),
    )(ue_sumT, bcntT, uv_sumT, ncntT, upT, uinT,
      w1, b1.reshape(-1, 1), w2, b2.reshape(-1, 1), w3, b3.reshape(-1, 1))


def _s2s_kernel(v_ref, b_ref, m_ref, q_ref, den_ref, r_ref):
    first = pl.program_id(0) == 0
    v = v_ref[...]
    blk = v.shape[0]
    bb = b_ref[0, 0, :]
    onehot = (bb[:, None] == jax.lax.broadcasted_iota(
        jnp.int32, (blk, NUM_GRAPHS), 1)).astype(jnp.float32)
    mm = m_ref[...]
    mm = jnp.where(mm > -1e30, mm, 0.0)
    ee = jax.lax.dot_general(v, q_ref[...], (((1,), (1,)), ((), ())),
                             preferred_element_type=jnp.float32)
    mrow = jax.lax.dot_general(onehot, mm, (((1,), (1,)), ((), ())),
                               preferred_element_type=jnp.float32)
    a = jnp.exp(ee - mrow)

    @pl.when(first)
    def _():
        den_ref[...] = jnp.zeros_like(den_ref)
        r_ref[...] = jnp.zeros_like(r_ref)

    den_ref[...] += jax.lax.dot_general(
        a, onehot, (((0,), (0,)), ((), ())),
        preferred_element_type=jnp.float32)
    r_ref[...] += _dgT(a * v, onehot)


def _s2s_pass2(v, batch3, m, q, blk):
    n = v.shape[0]
    n_blk = n // blk
    return pl.pallas_call(
        _s2s_kernel,
        grid=(n_blk,),
        in_specs=[_row_spec(blk, 32), _idx_spec(blk),
                  _const_spec((1, NUM_GRAPHS)), _const_spec((1, 32))],
        out_specs=[_const_spec((1, NUM_GRAPHS)),
                   _const_spec((32, NUM_GRAPHS))],
        out_shape=[jax.ShapeDtypeStruct((1, NUM_GRAPHS), jnp.float32),
                   jax.ShapeDtypeStruct((32, NUM_GRAPHS), jnp.float32)],
    )(v, batch3, m, q)


def _head_kernel(rv_ref, dv_ref, re_ref, de_ref, qv_ref, qe_ref, sT_ref,
                 w0_ref, b0_ref, w1_ref, b1_ref, w2_ref, b2_ref, o_ref):
    rv = rv_ref[...] / (dv_ref[...] + 1e-16)
    re = re_ref[...] / (de_ref[...] + 1e-16)
    g = rv.shape[1]
    qv = jnp.broadcast_to(qv_ref[...].reshape(32, 1), (32, g))
    qe = jnp.broadcast_to(qe_ref[...].reshape(32, 1), (32, g))
    tmp = jnp.concatenate([qv, rv, qe, re, sT_ref[...]], axis=0)
    h = _ssp(_matT(w0_ref[...], tmp, b0_ref[...]))
    h = _ssp(_matT(w1_ref[...], h, b1_ref[...]))
    o_ref[...] = _matT(w2_ref[...], h, b2_ref[...])


def _head(params, rvT, denvT, reT, deneT, qv, qe, stateT):
    return pl.pallas_call(
        _head_kernel,
        out_shape=jax.ShapeDtypeStruct((1, NUM_GRAPHS), jnp.float32),
    )(rvT, denvT, reT, deneT, qv, qe, stateT,
      params['h0']['W'], params['h0']['b'].reshape(-1, 1),
      params['h1']['W'], params['h1']['b'].reshape(-1, 1),
      params['h2']['W'], params['h2']['b'].reshape(-1, 1))


def _s2s_query(p):
    gates = p['b_ih'] + p['b_hh']
    i, f, g, o = jnp.split(gates, 4)
    c = jax.nn.sigmoid(i) * jnp.tanh(g)
    q = jax.nn.sigmoid(o) * jnp.tanh(c)
    return q.reshape(1, 32)


def kernel(x, edge_index, edge_attr, state, batch, bond_batch, params):
    p_all = params
    src, dst = edge_index[0], edge_index[1]
    n_nodes = x.shape[0]
    n_edges = edge_attr.shape[0]

    batch3 = batch.astype(jnp.int32).reshape(n_nodes // NODE_BLK, 1, NODE_BLK)
    bond3 = bond_batch.astype(jnp.int32).reshape(
        n_edges // EDGE_BLK, 1, EDGE_BLK)
    x3 = x.astype(jnp.int32).reshape(n_nodes // NODE_BLK, 1, NODE_BLK)

    ones = jnp.ones((n_edges,), jnp.float32)
    deg = jax.ops.segment_sum(ones, dst, num_segments=n_nodes)
    degc = jnp.maximum(deg, 1.0).reshape(n_nodes, 1)

    stateT = state.T
    qv = _s2s_query(p_all['sv'])
    qe = _s2s_query(p_all['se'])
    dummy_emb = p_all['emb']

    xv = None
    ea = edge_attr
    m_e = m_v = None

    for mi, (pname, skip) in enumerate(
            [('m1', True), ('m2', False), ('m3', False)]):
        p = p_all[pname]
        last = mi == 2
        upT = _pre_u(p, stateT)
        if mi == 0:
            x_p = _node_pre(p, x3, dummy_emb, True, n_nodes)
        else:
            x_p = _node_pre(p, xv, dummy_emb, False, n_nodes)
        xs = x_p[src]
        xd = x_p[dst]
        e_new, e_out, ue_sumT, bcntT, m_e_k = _edge_pipeline(
            p, xs, xd, ea, bond3, upT, qe, skip, last)
        agg_sum = jax.ops.segment_sum(e_new, dst, num_segments=n_nodes)
        x_in = x_p if skip else xv
        v_out, uv_sumT, ncntT, m_v_k = _node_post(
            p, agg_sum, degc, x_p, x_in, batch3, upT, qv, last)
        u_inT = upT if skip else stateT
        stateT = _phi_u(p, ue_sumT, bcntT, uv_sumT, ncntT, upT, u_inT)
        xv = v_out
        ea = e_out
        if last:
            m_e, m_v = m_e_k, m_v_k

    denvT, rvT = _s2s_pass2(xv, batch3, m_v, qv, NODE_BLK)
    deneT, reT = _s2s_pass2(ea, bond3, m_e, qe, EDGE_BLK)
    out = _head(p_all, rvT, denvT, reT, deneT, qv, qe, stateT)
    return out.reshape(NUM_GRAPHS, 1)


# speedup vs baseline: 3.2755x; 1.5542x over previous
import functools

import jax
import jax.numpy as jnp
from jax import lax
from jax.experimental import pallas as pl
from jax.experimental.pallas import tpu as pltpu
from jax.experimental.pallas import tpu_sc as plsc

NUM_GRAPHS = 512
EDGE_BLK = 2000
NODE_BLK = 2000
_LOG2 = 0.6931471805599453


def _ssp(x):
    return jnp.maximum(x, 0.0) + jnp.log1p(jnp.exp(-jnp.abs(x))) - _LOG2


def _dg(x, w):
    return jax.lax.dot_general(x, w, (((1,), (1,)), ((), ())),
                               preferred_element_type=jnp.float32)


def _dgT(a, b):
    return jax.lax.dot_general(a, b, (((0,), (0,)), ((), ())),
                               preferred_element_type=jnp.float32)


def _matT(w, xT, b):
    return jax.lax.dot_general(w, xT, (((1,), (0,)), ((), ())),
                               preferred_element_type=jnp.float32) + b


def _const_spec(shape):
    return pl.BlockSpec(shape, lambda n: tuple(0 for _ in shape))


def _row_spec(blk, feat):
    return pl.BlockSpec((blk, feat), lambda n: (n, 0))


def _idx_spec(blk):
    return pl.BlockSpec((1, 1, blk), lambda n: (n, 0, 0))


_SC_NW = 32
_GCHUNK = 2000


def _sc_gather_body(table_hbm, idx_hbm, out_hbm, idx_v, rows_v, sem,
                    *, b_per_w, n_ch):
    wid = lax.axis_index("s") * 2 + lax.axis_index("c")
    base = wid * b_per_w

    def body(i, carry):
        off = base + i * _GCHUNK
        pltpu.sync_copy(idx_hbm.at[pl.ds(off, _GCHUNK)], idx_v)
        pltpu.async_copy(table_hbm.at[idx_v], rows_v, sem).wait()
        pltpu.sync_copy(rows_v, out_hbm.at[pl.ds(off, _GCHUNK)])
        return carry

    lax.fori_loop(0, n_ch, body, 0)


def _sc_gather(table, idx):
    B = idx.shape[0]
    D = table.shape[1]
    b_per_w = B // _SC_NW
    n_ch = b_per_w // _GCHUNK
    assert b_per_w % _GCHUNK == 0
    mesh = plsc.VectorSubcoreMesh(core_axis_name="c", subcore_axis_name="s")
    fn = functools.partial(_sc_gather_body, b_per_w=b_per_w, n_ch=n_ch)
    return pl.kernel(
        fn, mesh=mesh,
        compiler_params=pltpu.CompilerParams(use_tc_tiling_on_sc=False),
        out_type=jax.ShapeDtypeStruct((B, D), jnp.float32),
        scratch_types=[
            pltpu.VMEM((_GCHUNK,), jnp.int32),
            pltpu.VMEM((_GCHUNK, D), jnp.float32),
            pltpu.SemaphoreType.DMA,
        ],
    )(table, idx)


def _edge_kernel(xs_ref, xd_ref, ea_ref, bb_ref, upT_ref, qe_ref,
                 w1_ref, b1_ref, w2_ref, b2_ref,
                 v1_ref, c1_ref, v2_ref, c2_ref, v3_ref, c3_ref,
                 enew_ref, eout_ref, ue_ref, bcnt_ref, me_ref,
                 *, skip_is_ep, compute_max):
    first = pl.program_id(0) == 0
    ea = ea_ref[...]
    e_p = _ssp(_dg(ea, w1_ref[...]) + b1_ref[...])
    e_p = _ssp(_dg(e_p, w2_ref[...]) + b2_ref[...])

    blk = ea.shape[0]
    bb = bb_ref[0, 0, :]
    onehot = (bb[:, None] == jax.lax.broadcasted_iota(
        jnp.int32, (blk, NUM_GRAPHS), 1)).astype(jnp.float32)
    ub = jax.lax.dot_general(onehot, upT_ref[...], (((1,), (1,)), ((), ())),
                             preferred_element_type=jnp.float32)

    h = jnp.concatenate([xs_ref[...], xd_ref[...], e_p, ub], axis=1)
    h = _ssp(_dg(h, v1_ref[...]) + c1_ref[...])
    h = _ssp(_dg(h, v2_ref[...]) + c2_ref[...])
    e_new = _ssp(_dg(h, v3_ref[...]) + c3_ref[...])
    enew_ref[...] = e_new
    if skip_is_ep:
        e_out = e_new + e_p
    else:
        e_out = e_new + ea
    eout_ref[...] = e_out

    @pl.when(first)
    def _():
        ue_ref[...] = jnp.zeros_like(ue_ref)
        bcnt_ref[...] = jnp.zeros_like(bcnt_ref)

    ue_ref[...] += _dgT(e_new, onehot)
    bcnt_ref[...] += jnp.sum(onehot, axis=0, keepdims=True)

    if compute_max:
        ee = jax.lax.dot_general(e_out, qe_ref[...], (((1,), (1,)), ((), ())),
                                 preferred_element_type=jnp.float32)
        masked = jnp.where(onehot > 0.5, ee, -jnp.inf)
        m_part = jnp.max(masked, axis=0, keepdims=True)

        @pl.when(first)
        def _():
            me_ref[...] = jnp.full_like(me_ref, -jnp.inf)

        me_ref[...] = jnp.maximum(me_ref[...], m_part)


def _edge_pipeline(p, xs, xd, edge_attr, bond3, upT, qe, skip_is_ep,
                   compute_max):
    E = xs.shape[0]
    n_blk = E // EDGE_BLK
    w1, b1 = p['pre_e'][0]['W'], p['pre_e'][0]['b']
    w2, b2 = p['pre_e'][1]['W'], p['pre_e'][1]['b']
    v1, c1 = p['phi_e'][0]['W'], p['phi_e'][0]['b']
    v2, c2 = p['phi_e'][1]['W'], p['phi_e'][1]['b']
    v3, c3 = p['phi_e'][2]['W'], p['phi_e'][2]['b']
    e_dim = edge_attr.shape[1]

    out = pl.pallas_call(
        functools.partial(_edge_kernel, skip_is_ep=skip_is_ep,
                          compute_max=compute_max),
        grid=(n_blk,),
        in_specs=[
            _row_spec(EDGE_BLK, 32), _row_spec(EDGE_BLK, 32),
            _row_spec(EDGE_BLK, e_dim), _idx_spec(EDGE_BLK),
            _const_spec((32, NUM_GRAPHS)), _const_spec((1, 32)),
            _const_spec(w1.shape), _const_spec((1, 64)),
            _const_spec(w2.shape), _const_spec((1, 32)),
            _const_spec(v1.shape), _const_spec((1, 64)),
            _const_spec(v2.shape), _const_spec((1, 64)),
            _const_spec(v3.shape), _const_spec((1, 32)),
        ],
        out_specs=[
            _row_spec(EDGE_BLK, 32), _row_spec(EDGE_BLK, 32),
            _const_spec((32, NUM_GRAPHS)), _const_spec((1, NUM_GRAPHS)),
            _const_spec((1, NUM_GRAPHS)),
        ],
        out_shape=[
            jax.ShapeDtypeStruct((E, 32), jnp.float32),
            jax.ShapeDtypeStruct((E, 32), jnp.float32),
            jax.ShapeDtypeStruct((32, NUM_GRAPHS), jnp.float32),
            jax.ShapeDtypeStruct((1, NUM_GRAPHS), jnp.float32),
            jax.ShapeDtypeStruct((1, NUM_GRAPHS), jnp.float32),
        ],
    )(xs, xd, edge_attr, bond3, upT, qe,
      w1, b1.reshape(1, -1), w2, b2.reshape(1, -1),
      v1, c1.reshape(1, -1), v2, c2.reshape(1, -1), v3, c3.reshape(1, -1))
    return out


def _node_pre_kernel(xv_ref, emb_ref, w1_ref, b1_ref, w2_ref, b2_ref,
                     xp_ref, *, with_emb):
    if with_emb:
        xi = xv_ref[0, 0, :]
        blk = xi.shape[0]
        oh = (xi[:, None] == jax.lax.broadcasted_iota(
            jnp.int32, (blk, 95), 1)).astype(jnp.float32)
        xv = jax.lax.dot_general(oh, emb_ref[...], (((1,), (0,)), ((), ())),
                                 preferred_element_type=jnp.float32)
    else:
        xv = xv_ref[...]
    h = _ssp(_dg(xv, w1_ref[...]) + b1_ref[...])
    xp_ref[...] = _ssp(_dg(h, w2_ref[...]) + b2_ref[...])


def _node_pre(p, xv_or_ids, emb, with_emb, n_nodes):
    n_blk = n_nodes // NODE_BLK
    w1, b1 = p['pre_v'][0]['W'], p['pre_v'][0]['b']
    w2, b2 = p['pre_v'][1]['W'], p['pre_v'][1]['b']
    if with_emb:
        first_spec = _idx_spec(NODE_BLK)
    else:
        first_spec = _row_spec(NODE_BLK, xv_or_ids.shape[1])
    return pl.pallas_call(
        functools.partial(_node_pre_kernel, with_emb=with_emb),
        grid=(n_blk,),
        in_specs=[first_spec, _const_spec(emb.shape),
                  _const_spec(w1.shape), _const_spec((1, 64)),
                  _const_spec(w2.shape), _const_spec((1, 32))],
        out_specs=_row_spec(NODE_BLK, 32),
        out_shape=jax.ShapeDtypeStruct((n_nodes, 32), jnp.float32),
    )(xv_or_ids, emb, w1, b1.reshape(1, -1), w2, b2.reshape(1, -1))


def _node_post_kernel(aggs_ref, deg_ref, xp_ref, xin_ref, b_ref, upT_ref,
                      qv_ref, w1_ref, b1_ref, w2_ref, b2_ref, w3_ref, b3_ref,
                      vout_ref, uv_ref, ncnt_ref, mv_ref, *, compute_max):
    first = pl.program_id(0) == 0
    agg = aggs_ref[...] / deg_ref[...]
    blk = agg.shape[0]
    bb = b_ref[0, 0, :]
    onehot = (bb[:, None] == jax.lax.broadcasted_iota(
        jnp.int32, (blk, NUM_GRAPHS), 1)).astype(jnp.float32)
    ub = jax.lax.dot_general(onehot, upT_ref[...], (((1,), (1,)), ((), ())),
                             preferred_element_type=jnp.float32)
    h = jnp.concatenate([agg, xp_ref[...], ub], axis=1)
    h = _ssp(_dg(h, w1_ref[...]) + b1_ref[...])
    h = _ssp(_dg(h, w2_ref[...]) + b2_ref[...])
    v_new = _ssp(_dg(h, w3_ref[...]) + b3_ref[...])
    v_out = v_new + xin_ref[...]
    vout_ref[...] = v_out

    @pl.when(first)
    def _():
        uv_ref[...] = jnp.zeros_like(uv_ref)
        ncnt_ref[...] = jnp.zeros_like(ncnt_ref)

    uv_ref[...] += _dgT(v_new, onehot)
    ncnt_ref[...] += jnp.sum(onehot, axis=0, keepdims=True)

    if compute_max:
        ev = jax.lax.dot_general(v_out, qv_ref[...], (((1,), (1,)), ((), ())),
                                 preferred_element_type=jnp.float32)
        masked = jnp.where(onehot > 0.5, ev, -jnp.inf)
        m_part = jnp.max(masked, axis=0, keepdims=True)

        @pl.when(first)
        def _():
            mv_ref[...] = jnp.full_like(mv_ref, -jnp.inf)

        mv_ref[...] = jnp.maximum(mv_ref[...], m_part)


def _node_post(p, agg_sum, degc, x_p, x_in, batch3, upT, qv, compute_max):
    n_nodes = x_p.shape[0]
    n_blk = n_nodes // NODE_BLK
    w1, b1 = p['phi_v'][0]['W'], p['phi_v'][0]['b']
    w2, b2 = p['phi_v'][1]['W'], p['phi_v'][1]['b']
    w3, b3 = p['phi_v'][2]['W'], p['phi_v'][2]['b']
    return pl.pallas_call(
        functools.partial(_node_post_kernel, compute_max=compute_max),
        grid=(n_blk,),
        in_specs=[
            _row_spec(NODE_BLK, 32), _row_spec(NODE_BLK, 1),
            _row_spec(NODE_BLK, 32), _row_spec(NODE_BLK, 32),
            _idx_spec(NODE_BLK), _const_spec((32, NUM_GRAPHS)),
            _const_spec((1, 32)),
            _const_spec(w1.shape), _const_spec((1, 64)),
            _const_spec(w2.shape), _const_spec((1, 64)),
            _const_spec(w3.shape), _const_spec((1, 32)),
        ],
        out_specs=[
            _row_spec(NODE_BLK, 32),
            _const_spec((32, NUM_GRAPHS)), _const_spec((1, NUM_GRAPHS)),
            _const_spec((1, NUM_GRAPHS)),
        ],
        out_shape=[
            jax.ShapeDtypeStruct((n_nodes, 32), jnp.float32),
            jax.ShapeDtypeStruct((32, NUM_GRAPHS), jnp.float32),
            jax.ShapeDtypeStruct((1, NUM_GRAPHS), jnp.float32),
            jax.ShapeDtypeStruct((1, NUM_GRAPHS), jnp.float32),
        ],
    )(agg_sum, degc, x_p, x_in, batch3, upT, qv,
      w1, b1.reshape(1, -1), w2, b2.reshape(1, -1), w3, b3.reshape(1, -1))


def _pre_u_kernel(sT_ref, w1_ref, b1_ref, w2_ref, b2_ref, o_ref):
    h = _ssp(_matT(w1_ref[...], sT_ref[...], b1_ref[...]))
    o_ref[...] = _ssp(_matT(w2_ref[...], h, b2_ref[...]))


def _pre_u(p, stateT):
    w1, b1 = p['pre_u'][0]['W'], p['pre_u'][0]['b']
    w2, b2 = p['pre_u'][1]['W'], p['pre_u'][1]['b']
    return pl.pallas_call(
        _pre_u_kernel,
        out_shape=jax.ShapeDtypeStruct((32, NUM_GRAPHS), jnp.float32),
    )(stateT, w1, b1.reshape(-1, 1), w2, b2.reshape(-1, 1))


def _phi_u_kernel(ue_ref, bcnt_ref, uv_ref, ncnt_ref, upT_ref, uinT_ref,
                  w1_ref, b1_ref, w2_ref, b2_ref, w3_ref, b3_ref, o_ref):
    u_e = ue_ref[...] / jnp.maximum(bcnt_ref[...], 1.0)
    u_v = uv_ref[...] / jnp.maximum(ncnt_ref[...], 1.0)
    h = jnp.concatenate([u_e, u_v, upT_ref[...]], axis=0)
    h = _ssp(_matT(w1_ref[...], h, b1_ref[...]))
    h = _ssp(_matT(w2_ref[...], h, b2_ref[...]))
    u_new = _ssp(_matT(w3_ref[...], h, b3_ref[...]))
    o_ref[...] = u_new + uinT_ref[...]


def _phi_u(p, ue_sumT, bcntT, uv_sumT, ncntT, upT, uinT):
    w1, b1 = p['phi_u'][0]['W'], p['phi_u'][0]['b']
    w2, b2 = p['phi_u'][1]['W'], p['phi_u'][1]['b']
    w3, b3 = p['phi_u'][2]['W'], p['phi_u'][2]['b']
    return pl.pallas_call(
        _phi_u_kernel,
        out_shape=jax.ShapeDtypeStruct((32, NUM_GRAPHS), jnp.float32),
    )(ue_sumT, bcntT, uv_sumT, ncntT, upT, uinT,
      w1, b1.reshape(-1, 1), w2, b2.reshape(-1, 1), w3, b3.reshape(-1, 1))


def _s2s_kernel(v_ref, b_ref, m_ref, q_ref, den_ref, r_ref):
    first = pl.program_id(0) == 0
    v = v_ref[...]
    blk = v.shape[0]
    bb = b_ref[0, 0, :]
    onehot = (bb[:, None] == jax.lax.broadcasted_iota(
        jnp.int32, (blk, NUM_GRAPHS), 1)).astype(jnp.float32)
    mm = m_ref[...]
    mm = jnp.where(mm > -1e30, mm, 0.0)
    ee = jax.lax.dot_general(v, q_ref[...], (((1,), (1,)), ((), ())),
                             preferred_element_type=jnp.float32)
    mrow = jax.lax.dot_general(onehot, mm, (((1,), (1,)), ((), ())),
                               preferred_element_type=jnp.float32)
    a = jnp.exp(ee - mrow)

    @pl.when(first)
    def _():
        den_ref[...] = jnp.zeros_like(den_ref)
        r_ref[...] = jnp.zeros_like(r_ref)

    den_ref[...] += jax.lax.dot_general(
        a, onehot, (((0,), (0,)), ((), ())),
        preferred_element_type=jnp.float32)
    r_ref[...] += _dgT(a * v, onehot)


def _s2s_pass2(v, batch3, m, q, blk):
    n = v.shape[0]
    n_blk = n // blk
    return pl.pallas_call(
        _s2s_kernel,
        grid=(n_blk,),
        in_specs=[_row_spec(blk, 32), _idx_spec(blk),
                  _const_spec((1, NUM_GRAPHS)), _const_spec((1, 32))],
        out_specs=[_const_spec((1, NUM_GRAPHS)),
                   _const_spec((32, NUM_GRAPHS))],
        out_shape=[jax.ShapeDtypeStruct((1, NUM_GRAPHS), jnp.float32),
                   jax.ShapeDtypeStruct((32, NUM_GRAPHS), jnp.float32)],
    )(v, batch3, m, q)


def _head_kernel(rv_ref, dv_ref, re_ref, de_ref, qv_ref, qe_ref, sT_ref,
                 w0_ref, b0_ref, w1_ref, b1_ref, w2_ref, b2_ref, o_ref):
    rv = rv_ref[...] / (dv_ref[...] + 1e-16)
    re = re_ref[...] / (de_ref[...] + 1e-16)
    g = rv.shape[1]
    qv = jnp.broadcast_to(qv_ref[...].reshape(32, 1), (32, g))
    qe = jnp.broadcast_to(qe_ref[...].reshape(32, 1), (32, g))
    tmp = jnp.concatenate([qv, rv, qe, re, sT_ref[...]], axis=0)
    h = _ssp(_matT(w0_ref[...], tmp, b0_ref[...]))
    h = _ssp(_matT(w1_ref[...], h, b1_ref[...]))
    o_ref[...] = _matT(w2_ref[...], h, b2_ref[...])


def _head(params, rvT, denvT, reT, deneT, qv, qe, stateT):
    return pl.pallas_call(
        _head_kernel,
        out_shape=jax.ShapeDtypeStruct((1, NUM_GRAPHS), jnp.float32),
    )(rvT, denvT, reT, deneT, qv, qe, stateT,
      params['h0']['W'], params['h0']['b'].reshape(-1, 1),
      params['h1']['W'], params['h1']['b'].reshape(-1, 1),
      params['h2']['W'], params['h2']['b'].reshape(-1, 1))


def _s2s_query(p):
    gates = p['b_ih'] + p['b_hh']
    i, f, g, o = jnp.split(gates, 4)
    c = jax.nn.sigmoid(i) * jnp.tanh(g)
    q = jax.nn.sigmoid(o) * jnp.tanh(c)
    return q.reshape(1, 32)


def kernel(x, edge_index, edge_attr, state, batch, bond_batch, params):
    p_all = params
    src = edge_index[0].astype(jnp.int32)
    dst = edge_index[1].astype(jnp.int32)
    n_nodes = x.shape[0]
    n_edges = edge_attr.shape[0]

    batch3 = batch.astype(jnp.int32).reshape(n_nodes // NODE_BLK, 1, NODE_BLK)
    bond3 = bond_batch.astype(jnp.int32).reshape(
        n_edges // EDGE_BLK, 1, EDGE_BLK)
    x3 = x.astype(jnp.int32).reshape(n_nodes // NODE_BLK, 1, NODE_BLK)

    ones = jnp.ones((n_edges,), jnp.float32)
    deg = jax.ops.segment_sum(ones, dst, num_segments=n_nodes)
    degc = jnp.maximum(deg, 1.0).reshape(n_nodes, 1)

    stateT = state.T
    qv = _s2s_query(p_all['sv'])
    qe = _s2s_query(p_all['se'])
    dummy_emb = p_all['emb']

    xv = None
    ea = edge_attr
    m_e = m_v = None

    for mi, (pname, skip) in enumerate(
            [('m1', True), ('m2', False), ('m3', False)]):
        p = p_all[pname]
        last = mi == 2
        upT = _pre_u(p, stateT)
        if mi == 0:
            x_p = _node_pre(p, x3, dummy_emb, True, n_nodes)
        else:
            x_p = _node_pre(p, xv, dummy_emb, False, n_nodes)
        xs = _sc_gather(x_p, src)
        xd = _sc_gather(x_p, dst)
        e_new, e_out, ue_sumT, bcntT, m_e_k = _edge_pipeline(
            p, xs, xd, ea, bond3, upT, qe, skip, last)
        agg_sum = jax.ops.segment_sum(e_new, dst, num_segments=n_nodes)
        x_in = x_p if skip else xv
        v_out, uv_sumT, ncntT, m_v_k = _node_post(
            p, agg_sum, degc, x_p, x_in, batch3, upT, qv, last)
        u_inT = upT if skip else stateT
        stateT = _phi_u(p, ue_sumT, bcntT, uv_sumT, ncntT, upT, u_inT)
        xv = v_out
        ea = e_out
        if last:
            m_e, m_v = m_e_k, m_v_k

    denvT, rvT = _s2s_pass2(xv, batch3, m_v, qv, NODE_BLK)
    deneT, reT = _s2s_pass2(ea, bond3, m_e, qe, EDGE_BLK)
    out = _head(p_all, rvT, denvT, reT, deneT, qv, qe, stateT)
    return out.reshape(NUM_GRAPHS, 1)


# speedup vs baseline: 4.0469x; 1.2355x over previous
import functools

import jax
import jax.numpy as jnp
from jax import lax
from jax.experimental import pallas as pl
from jax.experimental.pallas import tpu as pltpu
from jax.experimental.pallas import tpu_sc as plsc

NUM_GRAPHS = 512
EDGE_BLK = 2000
NODE_BLK = 2000
_LOG2 = 0.6931471805599453


def _ssp(x):
    return jnp.maximum(x, 0.0) + jnp.log1p(jnp.exp(-jnp.abs(x))) - _LOG2


def _dg(x, w):
    return jax.lax.dot_general(x, w, (((1,), (1,)), ((), ())),
                               preferred_element_type=jnp.float32)


def _dgT(a, b):
    return jax.lax.dot_general(a, b, (((0,), (0,)), ((), ())),
                               preferred_element_type=jnp.float32)


def _matT(w, xT, b):
    return jax.lax.dot_general(w, xT, (((1,), (0,)), ((), ())),
                               preferred_element_type=jnp.float32) + b


def _const_spec(shape):
    return pl.BlockSpec(shape, lambda n: tuple(0 for _ in shape))


def _row_spec(blk, feat):
    return pl.BlockSpec((blk, feat), lambda n: (n, 0))


def _idx_spec(blk):
    return pl.BlockSpec((1, 1, blk), lambda n: (n, 0, 0))


_SC_NW = 32
_GCHUNK = 2000


def _sc_gather_body(table_hbm, idx_hbm, out_hbm, idx_v, rows_v, sem,
                    *, b_per_w, n_ch):
    wid = lax.axis_index("s") * 2 + lax.axis_index("c")
    base = wid * b_per_w

    def body(i, carry):
        off = base + i * _GCHUNK
        pltpu.sync_copy(idx_hbm.at[pl.ds(off, _GCHUNK)], idx_v)
        pltpu.async_copy(table_hbm.at[idx_v], rows_v, sem).wait()
        pltpu.sync_copy(rows_v, out_hbm.at[pl.ds(off, _GCHUNK)])
        return carry

    lax.fori_loop(0, n_ch, body, 0)


def _sc_gather(table, idx):
    B = idx.shape[0]
    D = table.shape[1]
    b_per_w = B // _SC_NW
    n_ch = b_per_w // _GCHUNK
    assert b_per_w % _GCHUNK == 0
    mesh = plsc.VectorSubcoreMesh(core_axis_name="c", subcore_axis_name="s")
    fn = functools.partial(_sc_gather_body, b_per_w=b_per_w, n_ch=n_ch)
    return pl.kernel(
        fn, mesh=mesh,
        compiler_params=pltpu.CompilerParams(use_tc_tiling_on_sc=False),
        out_type=jax.ShapeDtypeStruct((B, D), jnp.float32),
        scratch_types=[
            pltpu.VMEM((_GCHUNK,), jnp.int32),
            pltpu.VMEM((_GCHUNK, D), jnp.float32),
            pltpu.SemaphoreType.DMA,
        ],
    )(table, idx)


_SCHUNK = 1000
_ACC_ROWS = 100096
_ZROWS = 6256


def _sc_scatter_body(vals_hbm, idx_hbm, zeros_hbm, out_hbm, acc, idx_v,
                     vals_v, *, n_edges, n_nodes):
    cid = lax.axis_index("c")
    sid = lax.axis_index("s")
    per_tile = n_edges // 16
    n_ch = per_tile // _SCHUNK
    last = n_nodes - 15 * _ZROWS

    pltpu.sync_copy(zeros_hbm, acc.at[pl.ds(sid * _ZROWS, _ZROWS)])
    plsc.subcore_barrier()

    def body(i, carry):
        off = sid * per_tile + i * _SCHUNK
        pltpu.sync_copy(idx_hbm.at[pl.ds(off, _SCHUNK)], idx_v)
        pltpu.sync_copy(vals_hbm.at[cid, pl.ds(off, _SCHUNK)], vals_v)
        pltpu.sync_copy(vals_v, acc.at[idx_v], add=True)
        return carry

    lax.fori_loop(0, n_ch, body, 0)
    plsc.subcore_barrier()

    @pl.when(sid < 15)
    def _():
        pltpu.sync_copy(acc.at[pl.ds(sid * _ZROWS, _ZROWS)],
                        out_hbm.at[cid, pl.ds(sid * _ZROWS, _ZROWS)])

    @pl.when(sid == 15)
    def _():
        pltpu.sync_copy(acc.at[pl.ds(15 * _ZROWS, last)],
                        out_hbm.at[cid, pl.ds(15 * _ZROWS, last)])


def _sc_scatter(vals2, idx, zeros, n_nodes):
    E = idx.shape[0]
    mesh = plsc.VectorSubcoreMesh(core_axis_name="c", subcore_axis_name="s")
    fn = functools.partial(_sc_scatter_body, n_edges=E, n_nodes=n_nodes)
    return pl.kernel(
        fn, mesh=mesh,
        compiler_params=pltpu.CompilerParams(use_tc_tiling_on_sc=False),
        out_type=jax.ShapeDtypeStruct((2, n_nodes, 16), jnp.float32),
        scratch_types=[
            pltpu.VMEM_SHARED((_ACC_ROWS, 16), jnp.float32),
            pltpu.VMEM((_SCHUNK,), jnp.int32),
            pltpu.VMEM((_SCHUNK, 16), jnp.float32),
        ],
    )(vals2, idx, zeros)


def _edge_kernel(xs_ref, xd_ref, ea_ref, bb_ref, upT_ref, qe_ref,
                 w1_ref, b1_ref, w2_ref, b2_ref,
                 v1_ref, c1_ref, v2_ref, c2_ref, v3_ref, c3_ref,
                 enew_ref, eout_ref, ue_ref, bcnt_ref, me_ref,
                 *, skip_is_ep, compute_max):
    first = pl.program_id(0) == 0
    ea = ea_ref[...]
    e_p = _ssp(_dg(ea, w1_ref[...]) + b1_ref[...])
    e_p = _ssp(_dg(e_p, w2_ref[...]) + b2_ref[...])

    blk = ea.shape[0]
    bb = bb_ref[0, 0, :]
    onehot = (bb[:, None] == jax.lax.broadcasted_iota(
        jnp.int32, (blk, NUM_GRAPHS), 1)).astype(jnp.float32)
    ub = jax.lax.dot_general(onehot, upT_ref[...], (((1,), (1,)), ((), ())),
                             preferred_element_type=jnp.float32)

    h = jnp.concatenate([xs_ref[...], xd_ref[...], e_p, ub], axis=1)
    h = _ssp(_dg(h, v1_ref[...]) + c1_ref[...])
    h = _ssp(_dg(h, v2_ref[...]) + c2_ref[...])
    e_new = _ssp(_dg(h, v3_ref[...]) + c3_ref[...])
    enew_ref[0] = e_new[:, :16]
    enew_ref[1] = e_new[:, 16:]
    if skip_is_ep:
        e_out = e_new + e_p
    else:
        e_out = e_new + ea
    eout_ref[...] = e_out

    @pl.when(first)
    def _():
        ue_ref[...] = jnp.zeros_like(ue_ref)
        bcnt_ref[...] = jnp.zeros_like(bcnt_ref)

    ue_ref[...] += _dgT(e_new, onehot)
    bcnt_ref[...] += jnp.sum(onehot, axis=0, keepdims=True)

    if compute_max:
        ee = jax.lax.dot_general(e_out, qe_ref[...], (((1,), (1,)), ((), ())),
                                 preferred_element_type=jnp.float32)
        masked = jnp.where(onehot > 0.5, ee, -jnp.inf)
        m_part = jnp.max(masked, axis=0, keepdims=True)

        @pl.when(first)
        def _():
            me_ref[...] = jnp.full_like(me_ref, -jnp.inf)

        me_ref[...] = jnp.maximum(me_ref[...], m_part)


def _edge_pipeline(p, xs, xd, edge_attr, bond3, upT, qe, skip_is_ep,
                   compute_max):
    E = xs.shape[0]
    n_blk = E // EDGE_BLK
    w1, b1 = p['pre_e'][0]['W'], p['pre_e'][0]['b']
    w2, b2 = p['pre_e'][1]['W'], p['pre_e'][1]['b']
    v1, c1 = p['phi_e'][0]['W'], p['phi_e'][0]['b']
    v2, c2 = p['phi_e'][1]['W'], p['phi_e'][1]['b']
    v3, c3 = p['phi_e'][2]['W'], p['phi_e'][2]['b']
    e_dim = edge_attr.shape[1]

    out = pl.pallas_call(
        functools.partial(_edge_kernel, skip_is_ep=skip_is_ep,
                          compute_max=compute_max),
        grid=(n_blk,),
        in_specs=[
            _row_spec(EDGE_BLK, 32), _row_spec(EDGE_BLK, 32),
            _row_spec(EDGE_BLK, e_dim), _idx_spec(EDGE_BLK),
            _const_spec((32, NUM_GRAPHS)), _const_spec((1, 32)),
            _const_spec(w1.shape), _const_spec((1, 64)),
            _const_spec(w2.shape), _const_spec((1, 32)),
            _const_spec(v1.shape), _const_spec((1, 64)),
            _const_spec(v2.shape), _const_spec((1, 64)),
            _const_spec(v3.shape), _const_spec((1, 32)),
        ],
        out_specs=[
            pl.BlockSpec((2, EDGE_BLK, 16), lambda n: (0, n, 0)),
            _row_spec(EDGE_BLK, 32),
            _const_spec((32, NUM_GRAPHS)), _const_spec((1, NUM_GRAPHS)),
            _const_spec((1, NUM_GRAPHS)),
        ],
        out_shape=[
            jax.ShapeDtypeStruct((2, E, 16), jnp.float32),
            jax.ShapeDtypeStruct((E, 32), jnp.float32),
            jax.ShapeDtypeStruct((32, NUM_GRAPHS), jnp.float32),
            jax.ShapeDtypeStruct((1, NUM_GRAPHS), jnp.float32),
            jax.ShapeDtypeStruct((1, NUM_GRAPHS), jnp.float32),
        ],
    )(xs, xd, edge_attr, bond3, upT, qe,
      w1, b1.reshape(1, -1), w2, b2.reshape(1, -1),
      v1, c1.reshape(1, -1), v2, c2.reshape(1, -1), v3, c3.reshape(1, -1))
    return out


def _node_pre_kernel(xv_ref, emb_ref, w1_ref, b1_ref, w2_ref, b2_ref,
                     xp_ref, *, with_emb):
    if with_emb:
        xi = xv_ref[0, 0, :]
        blk = xi.shape[0]
        oh = (xi[:, None] == jax.lax.broadcasted_iota(
            jnp.int32, (blk, 95), 1)).astype(jnp.float32)
        xv = jax.lax.dot_general(oh, emb_ref[...], (((1,), (0,)), ((), ())),
                                 preferred_element_type=jnp.float32)
    else:
        xv = xv_ref[...]
    h = _ssp(_dg(xv, w1_ref[...]) + b1_ref[...])
    xp_ref[...] = _ssp(_dg(h, w2_ref[...]) + b2_ref[...])


def _node_pre(p, xv_or_ids, emb, with_emb, n_nodes):
    n_blk = n_nodes // NODE_BLK
    w1, b1 = p['pre_v'][0]['W'], p['pre_v'][0]['b']
    w2, b2 = p['pre_v'][1]['W'], p['pre_v'][1]['b']
    if with_emb:
        first_spec = _idx_spec(NODE_BLK)
    else:
        first_spec = _row_spec(NODE_BLK, xv_or_ids.shape[1])
    return pl.pallas_call(
        functools.partial(_node_pre_kernel, with_emb=with_emb),
        grid=(n_blk,),
        in_specs=[first_spec, _const_spec(emb.shape),
                  _const_spec(w1.shape), _const_spec((1, 64)),
                  _const_spec(w2.shape), _const_spec((1, 32))],
        out_specs=_row_spec(NODE_BLK, 32),
        out_shape=jax.ShapeDtypeStruct((n_nodes, 32), jnp.float32),
    )(xv_or_ids, emb, w1, b1.reshape(1, -1), w2, b2.reshape(1, -1))


def _node_post_kernel(aggs_ref, deg_ref, xp_ref, xin_ref, b_ref, upT_ref,
                      qv_ref, w1_ref, b1_ref, w2_ref, b2_ref, w3_ref, b3_ref,
                      vout_ref, uv_ref, ncnt_ref, mv_ref, *, compute_max):
    first = pl.program_id(0) == 0
    agg = jnp.concatenate([aggs_ref[0], aggs_ref[1]], axis=1) / deg_ref[...]
    blk = agg.shape[0]
    bb = b_ref[0, 0, :]
    onehot = (bb[:, None] == jax.lax.broadcasted_iota(
        jnp.int32, (blk, NUM_GRAPHS), 1)).astype(jnp.float32)
    ub = jax.lax.dot_general(onehot, upT_ref[...], (((1,), (1,)), ((), ())),
                             preferred_element_type=jnp.float32)
    h = jnp.concatenate([agg, xp_ref[...], ub], axis=1)
    h = _ssp(_dg(h, w1_ref[...]) + b1_ref[...])
    h = _ssp(_dg(h, w2_ref[...]) + b2_ref[...])
    v_new = _ssp(_dg(h, w3_ref[...]) + b3_ref[...])
    v_out = v_new + xin_ref[...]
    vout_ref[...] = v_out

    @pl.when(first)
    def _():
        uv_ref[...] = jnp.zeros_like(uv_ref)
        ncnt_ref[...] = jnp.zeros_like(ncnt_ref)

    uv_ref[...] += _dgT(v_new, onehot)
    ncnt_ref[...] += jnp.sum(onehot, axis=0, keepdims=True)

    if compute_max:
        ev = jax.lax.dot_general(v_out, qv_ref[...], (((1,), (1,)), ((), ())),
                                 preferred_element_type=jnp.float32)
        masked = jnp.where(onehot > 0.5, ev, -jnp.inf)
        m_part = jnp.max(masked, axis=0, keepdims=True)

        @pl.when(first)
        def _():
            mv_ref[...] = jnp.full_like(mv_ref, -jnp.inf)

        mv_ref[...] = jnp.maximum(mv_ref[...], m_part)


def _node_post(p, agg_sum, degc, x_p, x_in, batch3, upT, qv, compute_max):
    n_nodes = x_p.shape[0]
    n_blk = n_nodes // NODE_BLK
    w1, b1 = p['phi_v'][0]['W'], p['phi_v'][0]['b']
    w2, b2 = p['phi_v'][1]['W'], p['phi_v'][1]['b']
    w3, b3 = p['phi_v'][2]['W'], p['phi_v'][2]['b']
    return pl.pallas_call(
        functools.partial(_node_post_kernel, compute_max=compute_max),
        grid=(n_blk,),
        in_specs=[
            pl.BlockSpec((2, NODE_BLK, 16), lambda n: (0, n, 0)),
            _row_spec(NODE_BLK, 1),
            _row_spec(NODE_BLK, 32), _row_spec(NODE_BLK, 32),
            _idx_spec(NODE_BLK), _const_spec((32, NUM_GRAPHS)),
            _const_spec((1, 32)),
            _const_spec(w1.shape), _const_spec((1, 64)),
            _const_spec(w2.shape), _const_spec((1, 64)),
            _const_spec(w3.shape), _const_spec((1, 32)),
        ],
        out_specs=[
            _row_spec(NODE_BLK, 32),
            _const_spec((32, NUM_GRAPHS)), _const_spec((1, NUM_GRAPHS)),
            _const_spec((1, NUM_GRAPHS)),
        ],
        out_shape=[
            jax.ShapeDtypeStruct((n_nodes, 32), jnp.float32),
            jax.ShapeDtypeStruct((32, NUM_GRAPHS), jnp.float32),
            jax.ShapeDtypeStruct((1, NUM_GRAPHS), jnp.float32),
            jax.ShapeDtypeStruct((1, NUM_GRAPHS), jnp.float32),
        ],
    )(agg_sum, degc, x_p, x_in, batch3, upT, qv,
      w1, b1.reshape(1, -1), w2, b2.reshape(1, -1), w3, b3.reshape(1, -1))


def _pre_u_kernel(sT_ref, w1_ref, b1_ref, w2_ref, b2_ref, o_ref):
    h = _ssp(_matT(w1_ref[...], sT_ref[...], b1_ref[...]))
    o_ref[...] = _ssp(_matT(w2_ref[...], h, b2_ref[...]))


def _pre_u(p, stateT):
    w1, b1 = p['pre_u'][0]['W'], p['pre_u'][0]['b']
    w2, b2 = p['pre_u'][1]['W'], p['pre_u'][1]['b']
    return pl.pallas_call(
        _pre_u_kernel,
        out_shape=jax.ShapeDtypeStruct((32, NUM_GRAPHS), jnp.float32),
    )(stateT, w1, b1.reshape(-1, 1), w2, b2.reshape(-1, 1))


def _phi_u_kernel(ue_ref, bcnt_ref, uv_ref, ncnt_ref, upT_ref, uinT_ref,
                  w1_ref, b1_ref, w2_ref, b2_ref, w3_ref, b3_ref, o_ref):
    u_e = ue_ref[...] / jnp.maximum(bcnt_ref[...], 1.0)
    u_v = uv_ref[...] / jnp.maximum(ncnt_ref[...], 1.0)
    h = jnp.concatenate([u_e, u_v, upT_ref[...]], axis=0)
    h = _ssp(_matT(w1_ref[...], h, b1_ref[...]))
    h = _ssp(_matT(w2_ref[...], h, b2_ref[...]))
    u_new = _ssp(_matT(w3_ref[...], h, b3_ref[...]))
    o_ref[...] = u_new + uinT_ref[...]


def _phi_u(p, ue_sumT, bcntT, uv_sumT, ncntT, upT, uinT):
    w1, b1 = p['phi_u'][0]['W'], p['phi_u'][0]['b']
    w2, b2 = p['phi_u'][1]['W'], p['phi_u'][1]['b']
    w3, b3 = p['phi_u'][2]['W'], p['phi_u'][2]['b']
    return pl.pallas_call(
        _phi_u_kernel,
        out_shape=jax.ShapeDtypeStruct((32, NUM_GRAPHS), jnp.float32),
    )(ue_sumT, bcntT, uv_sumT, ncntT, upT, uinT,
      w1, b1.reshape(-1, 1), w2, b2.reshape(-1, 1), w3, b3.reshape(-1, 1))


def _s2s_kernel(v_ref, b_ref, m_ref, q_ref, den_ref, r_ref):
    first = pl.program_id(0) == 0
    v = v_ref[...]
    blk = v.shape[0]
    bb = b_ref[0, 0, :]
    onehot = (bb[:, None] == jax.lax.broadcasted_iota(
        jnp.int32, (blk, NUM_GRAPHS), 1)).astype(jnp.float32)
    mm = m_ref[...]
    mm = jnp.where(mm > -1e30, mm, 0.0)
    ee = jax.lax.dot_general(v, q_ref[...], (((1,), (1,)), ((), ())),
                             preferred_element_type=jnp.float32)
    mrow = jax.lax.dot_general(onehot, mm, (((1,), (1,)), ((), ())),
                               preferred_element_type=jnp.float32)
    a = jnp.exp(ee - mrow)

    @pl.when(first)
    def _():
        den_ref[...] = jnp.zeros_like(den_ref)
        r_ref[...] = jnp.zeros_like(r_ref)

    den_ref[...] += jax.lax.dot_general(
        a, onehot, (((0,), (0,)), ((), ())),
        preferred_element_type=jnp.float32)
    r_ref[...] += _dgT(a * v, onehot)


def _s2s_pass2(v, batch3, m, q, blk):
    n = v.shape[0]
    n_blk = n // blk
    return pl.pallas_call(
        _s2s_kernel,
        grid=(n_blk,),
        in_specs=[_row_spec(blk, 32), _idx_spec(blk),
                  _const_spec((1, NUM_GRAPHS)), _const_spec((1, 32))],
        out_specs=[_const_spec((1, NUM_GRAPHS)),
                   _const_spec((32, NUM_GRAPHS))],
        out_shape=[jax.ShapeDtypeStruct((1, NUM_GRAPHS), jnp.float32),
                   jax.ShapeDtypeStruct((32, NUM_GRAPHS), jnp.float32)],
    )(v, batch3, m, q)


def _head_kernel(rv_ref, dv_ref, re_ref, de_ref, qv_ref, qe_ref, sT_ref,
                 w0_ref, b0_ref, w1_ref, b1_ref, w2_ref, b2_ref, o_ref):
    rv = rv_ref[...] / (dv_ref[...] + 1e-16)
    re = re_ref[...] / (de_ref[...] + 1e-16)
    g = rv.shape[1]
    qv = jnp.broadcast_to(qv_ref[...].reshape(32, 1), (32, g))
    qe = jnp.broadcast_to(qe_ref[...].reshape(32, 1), (32, g))
    tmp = jnp.concatenate([qv, rv, qe, re, sT_ref[...]], axis=0)
    h = _ssp(_matT(w0_ref[...], tmp, b0_ref[...]))
    h = _ssp(_matT(w1_ref[...], h, b1_ref[...]))
    o_ref[...] = _matT(w2_ref[...], h, b2_ref[...])


def _head(params, rvT, denvT, reT, deneT, qv, qe, stateT):
    return pl.pallas_call(
        _head_kernel,
        out_shape=jax.ShapeDtypeStruct((1, NUM_GRAPHS), jnp.float32),
    )(rvT, denvT, reT, deneT, qv, qe, stateT,
      params['h0']['W'], params['h0']['b'].reshape(-1, 1),
      params['h1']['W'], params['h1']['b'].reshape(-1, 1),
      params['h2']['W'], params['h2']['b'].reshape(-1, 1))


def _s2s_query(p):
    gates = p['b_ih'] + p['b_hh']
    i, f, g, o = jnp.split(gates, 4)
    c = jax.nn.sigmoid(i) * jnp.tanh(g)
    q = jax.nn.sigmoid(o) * jnp.tanh(c)
    return q.reshape(1, 32)


def kernel(x, edge_index, edge_attr, state, batch, bond_batch, params):
    p_all = params
    src = edge_index[0].astype(jnp.int32)
    dst = edge_index[1].astype(jnp.int32)
    n_nodes = x.shape[0]
    n_edges = edge_attr.shape[0]

    batch3 = batch.astype(jnp.int32).reshape(n_nodes // NODE_BLK, 1, NODE_BLK)
    bond3 = bond_batch.astype(jnp.int32).reshape(
        n_edges // EDGE_BLK, 1, EDGE_BLK)
    x3 = x.astype(jnp.int32).reshape(n_nodes // NODE_BLK, 1, NODE_BLK)

    ones = jnp.ones((n_edges,), jnp.float32)
    deg = jax.ops.segment_sum(ones, dst, num_segments=n_nodes)
    degc = jnp.maximum(deg, 1.0).reshape(n_nodes, 1)
    zeros = jnp.zeros((_ZROWS, 16), jnp.float32)

    stateT = state.T
    qv = _s2s_query(p_all['sv'])
    qe = _s2s_query(p_all['se'])
    dummy_emb = p_all['emb']

    xv = None
    ea = edge_attr
    m_e = m_v = None

    for mi, (pname, skip) in enumerate(
            [('m1', True), ('m2', False), ('m3', False)]):
        p = p_all[pname]
        last = mi == 2
        upT = _pre_u(p, stateT)
        if mi == 0:
            x_p = _node_pre(p, x3, dummy_emb, True, n_nodes)
        else:
            x_p = _node_pre(p, xv, dummy_emb, False, n_nodes)
        xs = _sc_gather(x_p, src)
        xd = _sc_gather(x_p, dst)
        e_new2, e_out, ue_sumT, bcntT, m_e_k = _edge_pipeline(
            p, xs, xd, ea, bond3, upT, qe, skip, last)
        agg2 = _sc_scatter(e_new2, dst, zeros, n_nodes)
        x_in = x_p if skip else xv
        v_out, uv_sumT, ncntT, m_v_k = _node_post(
            p, agg2, degc, x_p, x_in, batch3, upT, qv, last)
        u_inT = upT if skip else stateT
        stateT = _phi_u(p, ue_sumT, bcntT, uv_sumT, ncntT, upT, u_inT)
        xv = v_out
        ea = e_out
        if last:
            m_e, m_v = m_e_k, m_v_k

    denvT, rvT = _s2s_pass2(xv, batch3, m_v, qv, NODE_BLK)
    deneT, reT = _s2s_pass2(ea, bond3, m_e, qe, EDGE_BLK)
    out = _head(p_all, rvT, denvT, reT, deneT, qv, qe, stateT)
    return out.reshape(NUM_GRAPHS, 1)


# speedup vs baseline: 4.1880x; 1.0349x over previous
import functools

import jax
import jax.numpy as jnp
from jax import lax
from jax.experimental import pallas as pl
from jax.experimental.pallas import tpu as pltpu
from jax.experimental.pallas import tpu_sc as plsc

NUM_GRAPHS = 512
EDGE_BLK = 4000
NODE_BLK = 4000
_LOG2 = 0.6931471805599453


def _ssp(x):
    return jnp.maximum(x, 0.0) + jnp.log1p(jnp.exp(-jnp.abs(x))) - _LOG2


def _dg(x, w):
    return jax.lax.dot_general(x, w, (((1,), (1,)), ((), ())),
                               preferred_element_type=jnp.float32)


def _dgT(a, b):
    return jax.lax.dot_general(a, b, (((0,), (0,)), ((), ())),
                               preferred_element_type=jnp.float32)


def _matT(w, xT, b):
    return jax.lax.dot_general(w, xT, (((1,), (0,)), ((), ())),
                               preferred_element_type=jnp.float32) + b


def _const_spec(shape):
    return pl.BlockSpec(shape, lambda n: tuple(0 for _ in shape))


def _row_spec(blk, feat):
    return pl.BlockSpec((blk, feat), lambda n: (n, 0))


def _idx_spec(blk):
    return pl.BlockSpec((1, 1, blk), lambda n: (n, 0, 0))


_SC_NW = 32
_GCHUNK = 2000


def _sc_gather_body(table_hbm, idx_hbm, out_hbm, idx_v, rows_v, sem,
                    *, b_per_w, n_ch):
    wid = lax.axis_index("s") * 2 + lax.axis_index("c")
    base = wid * b_per_w

    def body(i, carry):
        off = base + i * _GCHUNK
        pltpu.sync_copy(idx_hbm.at[pl.ds(off, _GCHUNK)], idx_v)
        pltpu.async_copy(table_hbm.at[idx_v], rows_v, sem).wait()
        pltpu.sync_copy(rows_v, out_hbm.at[pl.ds(off, _GCHUNK)])
        return carry

    lax.fori_loop(0, n_ch, body, 0)


def _sc_gather(table, idx):
    B = idx.shape[0]
    D = table.shape[1]
    b_per_w = B // _SC_NW
    n_ch = b_per_w // _GCHUNK
    assert b_per_w % _GCHUNK == 0
    mesh = plsc.VectorSubcoreMesh(core_axis_name="c", subcore_axis_name="s")
    fn = functools.partial(_sc_gather_body, b_per_w=b_per_w, n_ch=n_ch)
    return pl.kernel(
        fn, mesh=mesh,
        compiler_params=pltpu.CompilerParams(use_tc_tiling_on_sc=False),
        out_type=jax.ShapeDtypeStruct((B, D), jnp.float32),
        scratch_types=[
            pltpu.VMEM((_GCHUNK,), jnp.int32),
            pltpu.VMEM((_GCHUNK, D), jnp.float32),
            pltpu.SemaphoreType.DMA,
        ],
    )(table, idx)


_SCHUNK = 1000
_ACC_ROWS = 100096
_ZROWS = 6256


def _sc_scatter_body(vals_hbm, idx_hbm, zeros_hbm, out_hbm, acc, idx_v,
                     vals_v, *, n_edges, n_nodes):
    cid = lax.axis_index("c")
    sid = lax.axis_index("s")
    per_tile = n_edges // 16
    n_ch = per_tile // _SCHUNK
    last = n_nodes - 15 * _ZROWS

    pltpu.sync_copy(zeros_hbm, acc.at[pl.ds(sid * _ZROWS, _ZROWS)])
    plsc.subcore_barrier()

    def body(i, carry):
        off = sid * per_tile + i * _SCHUNK
        pltpu.sync_copy(idx_hbm.at[pl.ds(off, _SCHUNK)], idx_v)
        pltpu.sync_copy(vals_hbm.at[cid, pl.ds(off, _SCHUNK)], vals_v)
        pltpu.sync_copy(vals_v, acc.at[idx_v], add=True)
        return carry

    lax.fori_loop(0, n_ch, body, 0)
    plsc.subcore_barrier()

    @pl.when(sid < 15)
    def _():
        pltpu.sync_copy(acc.at[pl.ds(sid * _ZROWS, _ZROWS)],
                        out_hbm.at[cid, pl.ds(sid * _ZROWS, _ZROWS)])

    @pl.when(sid == 15)
    def _():
        pltpu.sync_copy(acc.at[pl.ds(15 * _ZROWS, last)],
                        out_hbm.at[cid, pl.ds(15 * _ZROWS, last)])


def _sc_scatter(vals2, idx, zeros, n_nodes):
    E = idx.shape[0]
    mesh = plsc.VectorSubcoreMesh(core_axis_name="c", subcore_axis_name="s")
    fn = functools.partial(_sc_scatter_body, n_edges=E, n_nodes=n_nodes)
    return pl.kernel(
        fn, mesh=mesh,
        compiler_params=pltpu.CompilerParams(use_tc_tiling_on_sc=False),
        out_type=jax.ShapeDtypeStruct((2, n_nodes, 16), jnp.float32),
        scratch_types=[
            pltpu.VMEM_SHARED((_ACC_ROWS, 16), jnp.float32),
            pltpu.VMEM((_SCHUNK,), jnp.int32),
            pltpu.VMEM((_SCHUNK, 16), jnp.float32),
        ],
    )(vals2, idx, zeros)


def _edge_kernel(xs_ref, xd_ref, ea_ref, bb_ref, upT_ref, qe_ref,
                 w1_ref, b1_ref, w2_ref, b2_ref,
                 v1_ref, c1_ref, v2_ref, c2_ref, v3_ref, c3_ref,
                 enew_ref, eout_ref, ue_ref, bcnt_ref, me_ref,
                 *, skip_is_ep, compute_max):
    first = pl.program_id(0) == 0
    ea = ea_ref[...]
    e_p = _ssp(_dg(ea, w1_ref[...]) + b1_ref[...])
    e_p = _ssp(_dg(e_p, w2_ref[...]) + b2_ref[...])

    blk = ea.shape[0]
    bb = bb_ref[0, 0, :]
    onehot = (bb[:, None] == jax.lax.broadcasted_iota(
        jnp.int32, (blk, NUM_GRAPHS), 1)).astype(jnp.float32)
    ub = jax.lax.dot_general(onehot, upT_ref[...], (((1,), (1,)), ((), ())),
                             preferred_element_type=jnp.float32)

    h = jnp.concatenate([xs_ref[...], xd_ref[...], e_p, ub], axis=1)
    h = _ssp(_dg(h, v1_ref[...]) + c1_ref[...])
    h = _ssp(_dg(h, v2_ref[...]) + c2_ref[...])
    e_new = _ssp(_dg(h, v3_ref[...]) + c3_ref[...])
    enew_ref[0] = e_new[:, :16]
    enew_ref[1] = e_new[:, 16:]
    if skip_is_ep:
        e_out = e_new + e_p
    else:
        e_out = e_new + ea
    eout_ref[...] = e_out

    @pl.when(first)
    def _():
        ue_ref[...] = jnp.zeros_like(ue_ref)
        bcnt_ref[...] = jnp.zeros_like(bcnt_ref)

    ue_ref[...] += _dgT(e_new, onehot)
    bcnt_ref[...] += jnp.sum(onehot, axis=0, keepdims=True)

    if compute_max:
        ee = jax.lax.dot_general(e_out, qe_ref[...], (((1,), (1,)), ((), ())),
                                 preferred_element_type=jnp.float32)
        masked = jnp.where(onehot > 0.5, ee, -jnp.inf)
        m_part = jnp.max(masked, axis=0, keepdims=True)

        @pl.when(first)
        def _():
            me_ref[...] = jnp.full_like(me_ref, -jnp.inf)

        me_ref[...] = jnp.maximum(me_ref[...], m_part)


def _edge_pipeline(p, xs, xd, edge_attr, bond3, upT, qe, skip_is_ep,
                   compute_max):
    E = xs.shape[0]
    n_blk = E // EDGE_BLK
    w1, b1 = p['pre_e'][0]['W'], p['pre_e'][0]['b']
    w2, b2 = p['pre_e'][1]['W'], p['pre_e'][1]['b']
    v1, c1 = p['phi_e'][0]['W'], p['phi_e'][0]['b']
    v2, c2 = p['phi_e'][1]['W'], p['phi_e'][1]['b']
    v3, c3 = p['phi_e'][2]['W'], p['phi_e'][2]['b']
    e_dim = edge_attr.shape[1]

    out = pl.pallas_call(
        functools.partial(_edge_kernel, skip_is_ep=skip_is_ep,
                          compute_max=compute_max),
        grid=(n_blk,),
        in_specs=[
            _row_spec(EDGE_BLK, 32), _row_spec(EDGE_BLK, 32),
            _row_spec(EDGE_BLK, e_dim), _idx_spec(EDGE_BLK),
            _const_spec((32, NUM_GRAPHS)), _const_spec((1, 32)),
            _const_spec(w1.shape), _const_spec((1, 64)),
            _const_spec(w2.shape), _const_spec((1, 32)),
            _const_spec(v1.shape), _const_spec((1, 64)),
            _const_spec(v2.shape), _const_spec((1, 64)),
            _const_spec(v3.shape), _const_spec((1, 32)),
        ],
        out_specs=[
            pl.BlockSpec((2, EDGE_BLK, 16), lambda n: (0, n, 0)),
            _row_spec(EDGE_BLK, 32),
            _const_spec((32, NUM_GRAPHS)), _const_spec((1, NUM_GRAPHS)),
            _const_spec((1, NUM_GRAPHS)),
        ],
        out_shape=[
            jax.ShapeDtypeStruct((2, E, 16), jnp.float32),
            jax.ShapeDtypeStruct((E, 32), jnp.float32),
            jax.ShapeDtypeStruct((32, NUM_GRAPHS), jnp.float32),
            jax.ShapeDtypeStruct((1, NUM_GRAPHS), jnp.float32),
            jax.ShapeDtypeStruct((1, NUM_GRAPHS), jnp.float32),
        ],
    )(xs, xd, edge_attr, bond3, upT, qe,
      w1, b1.reshape(1, -1), w2, b2.reshape(1, -1),
      v1, c1.reshape(1, -1), v2, c2.reshape(1, -1), v3, c3.reshape(1, -1))
    return out


def _node_pre_kernel(xv_ref, emb_ref, w1_ref, b1_ref, w2_ref, b2_ref,
                     xp_ref, *, with_emb):
    if with_emb:
        xi = xv_ref[0, 0, :]
        blk = xi.shape[0]
        oh = (xi[:, None] == jax.lax.broadcasted_iota(
            jnp.int32, (blk, 95), 1)).astype(jnp.float32)
        xv = jax.lax.dot_general(oh, emb_ref[...], (((1,), (0,)), ((), ())),
                                 preferred_element_type=jnp.float32)
    else:
        xv = xv_ref[...]
    h = _ssp(_dg(xv, w1_ref[...]) + b1_ref[...])
    xp_ref[...] = _ssp(_dg(h, w2_ref[...]) + b2_ref[...])


def _node_pre(p, xv_or_ids, emb, with_emb, n_nodes):
    n_blk = n_nodes // NODE_BLK
    w1, b1 = p['pre_v'][0]['W'], p['pre_v'][0]['b']
    w2, b2 = p['pre_v'][1]['W'], p['pre_v'][1]['b']
    if with_emb:
        first_spec = _idx_spec(NODE_BLK)
    else:
        first_spec = _row_spec(NODE_BLK, xv_or_ids.shape[1])
    return pl.pallas_call(
        functools.partial(_node_pre_kernel, with_emb=with_emb),
        grid=(n_blk,),
        in_specs=[first_spec, _const_spec(emb.shape),
                  _const_spec(w1.shape), _const_spec((1, 64)),
                  _const_spec(w2.shape), _const_spec((1, 32))],
        out_specs=_row_spec(NODE_BLK, 32),
        out_shape=jax.ShapeDtypeStruct((n_nodes, 32), jnp.float32),
    )(xv_or_ids, emb, w1, b1.reshape(1, -1), w2, b2.reshape(1, -1))


def _node_post_kernel(aggs_ref, deg_ref, xp_ref, xin_ref, b_ref, upT_ref,
                      qv_ref, w1_ref, b1_ref, w2_ref, b2_ref, w3_ref, b3_ref,
                      vout_ref, uv_ref, ncnt_ref, mv_ref, *, compute_max):
    first = pl.program_id(0) == 0
    agg = jnp.concatenate([aggs_ref[0], aggs_ref[1]], axis=1) / deg_ref[...]
    blk = agg.shape[0]
    bb = b_ref[0, 0, :]
    onehot = (bb[:, None] == jax.lax.broadcasted_iota(
        jnp.int32, (blk, NUM_GRAPHS), 1)).astype(jnp.float32)
    ub = jax.lax.dot_general(onehot, upT_ref[...], (((1,), (1,)), ((), ())),
                             preferred_element_type=jnp.float32)
    h = jnp.concatenate([agg, xp_ref[...], ub], axis=1)
    h = _ssp(_dg(h, w1_ref[...]) + b1_ref[...])
    h = _ssp(_dg(h, w2_ref[...]) + b2_ref[...])
    v_new = _ssp(_dg(h, w3_ref[...]) + b3_ref[...])
    v_out = v_new + xin_ref[...]
    vout_ref[...] = v_out

    @pl.when(first)
    def _():
        uv_ref[...] = jnp.zeros_like(uv_ref)
        ncnt_ref[...] = jnp.zeros_like(ncnt_ref)

    uv_ref[...] += _dgT(v_new, onehot)
    ncnt_ref[...] += jnp.sum(onehot, axis=0, keepdims=True)

    if compute_max:
        ev = jax.lax.dot_general(v_out, qv_ref[...], (((1,), (1,)), ((), ())),
                                 preferred_element_type=jnp.float32)
        masked = jnp.where(onehot > 0.5, ev, -jnp.inf)
        m_part = jnp.max(masked, axis=0, keepdims=True)

        @pl.when(first)
        def _():
            mv_ref[...] = jnp.full_like(mv_ref, -jnp.inf)

        mv_ref[...] = jnp.maximum(mv_ref[...], m_part)


def _node_post(p, agg_sum, degc, x_p, x_in, batch3, upT, qv, compute_max):
    n_nodes = x_p.shape[0]
    n_blk = n_nodes // NODE_BLK
    w1, b1 = p['phi_v'][0]['W'], p['phi_v'][0]['b']
    w2, b2 = p['phi_v'][1]['W'], p['phi_v'][1]['b']
    w3, b3 = p['phi_v'][2]['W'], p['phi_v'][2]['b']
    return pl.pallas_call(
        functools.partial(_node_post_kernel, compute_max=compute_max),
        grid=(n_blk,),
        in_specs=[
            pl.BlockSpec((2, NODE_BLK, 16), lambda n: (0, n, 0)),
            _row_spec(NODE_BLK, 1),
            _row_spec(NODE_BLK, 32), _row_spec(NODE_BLK, 32),
            _idx_spec(NODE_BLK), _const_spec((32, NUM_GRAPHS)),
            _const_spec((1, 32)),
            _const_spec(w1.shape), _const_spec((1, 64)),
            _const_spec(w2.shape), _const_spec((1, 64)),
            _const_spec(w3.shape), _const_spec((1, 32)),
        ],
        out_specs=[
            _row_spec(NODE_BLK, 32),
            _const_spec((32, NUM_GRAPHS)), _const_spec((1, NUM_GRAPHS)),
            _const_spec((1, NUM_GRAPHS)),
        ],
        out_shape=[
            jax.ShapeDtypeStruct((n_nodes, 32), jnp.float32),
            jax.ShapeDtypeStruct((32, NUM_GRAPHS), jnp.float32),
            jax.ShapeDtypeStruct((1, NUM_GRAPHS), jnp.float32),
            jax.ShapeDtypeStruct((1, NUM_GRAPHS), jnp.float32),
        ],
    )(agg_sum, degc, x_p, x_in, batch3, upT, qv,
      w1, b1.reshape(1, -1), w2, b2.reshape(1, -1), w3, b3.reshape(1, -1))


def _pre_u_kernel(sT_ref, w1_ref, b1_ref, w2_ref, b2_ref, o_ref):
    h = _ssp(_matT(w1_ref[...], sT_ref[...], b1_ref[...]))
    o_ref[...] = _ssp(_matT(w2_ref[...], h, b2_ref[...]))


def _pre_u(p, stateT):
    w1, b1 = p['pre_u'][0]['W'], p['pre_u'][0]['b']
    w2, b2 = p['pre_u'][1]['W'], p['pre_u'][1]['b']
    return pl.pallas_call(
        _pre_u_kernel,
        out_shape=jax.ShapeDtypeStruct((32, NUM_GRAPHS), jnp.float32),
    )(stateT, w1, b1.reshape(-1, 1), w2, b2.reshape(-1, 1))


def _phi_u_kernel(ue_ref, bcnt_ref, uv_ref, ncnt_ref, upT_ref, uinT_ref,
                  w1_ref, b1_ref, w2_ref, b2_ref, w3_ref, b3_ref, o_ref):
    u_e = ue_ref[...] / jnp.maximum(bcnt_ref[...], 1.0)
    u_v = uv_ref[...] / jnp.maximum(ncnt_ref[...], 1.0)
    h = jnp.concatenate([u_e, u_v, upT_ref[...]], axis=0)
    h = _ssp(_matT(w1_ref[...], h, b1_ref[...]))
    h = _ssp(_matT(w2_ref[...], h, b2_ref[...]))
    u_new = _ssp(_matT(w3_ref[...], h, b3_ref[...]))
    o_ref[...] = u_new + uinT_ref[...]


def _phi_u(p, ue_sumT, bcntT, uv_sumT, ncntT, upT, uinT):
    w1, b1 = p['phi_u'][0]['W'], p['phi_u'][0]['b']
    w2, b2 = p['phi_u'][1]['W'], p['phi_u'][1]['b']
    w3, b3 = p['phi_u'][2]['W'], p['phi_u'][2]['b']
    return pl.pallas_call(
        _phi_u_kernel,
        out_shape=jax.ShapeDtypeStruct((32, NUM_GRAPHS), jnp.float32),
    )(ue_sumT, bcntT, uv_sumT, ncntT, upT, uinT,
      w1, b1.reshape(-1, 1), w2, b2.reshape(-1, 1), w3, b3.reshape(-1, 1))


def _s2s_kernel(v_ref, b_ref, m_ref, q_ref, den_ref, r_ref):
    first = pl.program_id(0) == 0
    v = v_ref[...]
    blk = v.shape[0]
    bb = b_ref[0, 0, :]
    onehot = (bb[:, None] == jax.lax.broadcasted_iota(
        jnp.int32, (blk, NUM_GRAPHS), 1)).astype(jnp.float32)
    mm = m_ref[...]
    mm = jnp.where(mm > -1e30, mm, 0.0)
    ee = jax.lax.dot_general(v, q_ref[...], (((1,), (1,)), ((), ())),
                             preferred_element_type=jnp.float32)
    mrow = jax.lax.dot_general(onehot, mm, (((1,), (1,)), ((), ())),
                               preferred_element_type=jnp.float32)
    a = jnp.exp(ee - mrow)

    @pl.when(first)
    def _():
        den_ref[...] = jnp.zeros_like(den_ref)
        r_ref[...] = jnp.zeros_like(r_ref)

    den_ref[...] += jax.lax.dot_general(
        a, onehot, (((0,), (0,)), ((), ())),
        preferred_element_type=jnp.float32)
    r_ref[...] += _dgT(a * v, onehot)


def _s2s_pass2(v, batch3, m, q, blk):
    n = v.shape[0]
    n_blk = n // blk
    return pl.pallas_call(
        _s2s_kernel,
        grid=(n_blk,),
        in_specs=[_row_spec(blk, 32), _idx_spec(blk),
                  _const_spec((1, NUM_GRAPHS)), _const_spec((1, 32))],
        out_specs=[_const_spec((1, NUM_GRAPHS)),
                   _const_spec((32, NUM_GRAPHS))],
        out_shape=[jax.ShapeDtypeStruct((1, NUM_GRAPHS), jnp.float32),
                   jax.ShapeDtypeStruct((32, NUM_GRAPHS), jnp.float32)],
    )(v, batch3, m, q)


def _head_kernel(rv_ref, dv_ref, re_ref, de_ref, qv_ref, qe_ref, sT_ref,
                 w0_ref, b0_ref, w1_ref, b1_ref, w2_ref, b2_ref, o_ref):
    rv = rv_ref[...] / (dv_ref[...] + 1e-16)
    re = re_ref[...] / (de_ref[...] + 1e-16)
    g = rv.shape[1]
    qv = jnp.broadcast_to(qv_ref[...].reshape(32, 1), (32, g))
    qe = jnp.broadcast_to(qe_ref[...].reshape(32, 1), (32, g))
    tmp = jnp.concatenate([qv, rv, qe, re, sT_ref[...]], axis=0)
    h = _ssp(_matT(w0_ref[...], tmp, b0_ref[...]))
    h = _ssp(_matT(w1_ref[...], h, b1_ref[...]))
    o_ref[...] = _matT(w2_ref[...], h, b2_ref[...])


def _head(params, rvT, denvT, reT, deneT, qv, qe, stateT):
    return pl.pallas_call(
        _head_kernel,
        out_shape=jax.ShapeDtypeStruct((1, NUM_GRAPHS), jnp.float32),
    )(rvT, denvT, reT, deneT, qv, qe, stateT,
      params['h0']['W'], params['h0']['b'].reshape(-1, 1),
      params['h1']['W'], params['h1']['b'].reshape(-1, 1),
      params['h2']['W'], params['h2']['b'].reshape(-1, 1))


def _s2s_query(p):
    gates = p['b_ih'] + p['b_hh']
    i, f, g, o = jnp.split(gates, 4)
    c = jax.nn.sigmoid(i) * jnp.tanh(g)
    q = jax.nn.sigmoid(o) * jnp.tanh(c)
    return q.reshape(1, 32)


def kernel(x, edge_index, edge_attr, state, batch, bond_batch, params):
    p_all = params
    src = edge_index[0].astype(jnp.int32)
    dst = edge_index[1].astype(jnp.int32)
    n_nodes = x.shape[0]
    n_edges = edge_attr.shape[0]

    batch3 = batch.astype(jnp.int32).reshape(n_nodes // NODE_BLK, 1, NODE_BLK)
    bond3 = bond_batch.astype(jnp.int32).reshape(
        n_edges // EDGE_BLK, 1, EDGE_BLK)
    x3 = x.astype(jnp.int32).reshape(n_nodes // NODE_BLK, 1, NODE_BLK)

    ones = jnp.ones((n_edges,), jnp.float32)
    deg = jax.ops.segment_sum(ones, dst, num_segments=n_nodes)
    degc = jnp.maximum(deg, 1.0).reshape(n_nodes, 1)
    zeros = jnp.zeros((_ZROWS, 16), jnp.float32)

    stateT = state.T
    qv = _s2s_query(p_all['sv'])
    qe = _s2s_query(p_all['se'])
    dummy_emb = p_all['emb']

    xv = None
    ea = edge_attr
    m_e = m_v = None

    for mi, (pname, skip) in enumerate(
            [('m1', True), ('m2', False), ('m3', False)]):
        p = p_all[pname]
        last = mi == 2
        upT = _pre_u(p, stateT)
        if mi == 0:
            x_p = _node_pre(p, x3, dummy_emb, True, n_nodes)
        else:
            x_p = _node_pre(p, xv, dummy_emb, False, n_nodes)
        xs = _sc_gather(x_p, src)
        xd = _sc_gather(x_p, dst)
        e_new2, e_out, ue_sumT, bcntT, m_e_k = _edge_pipeline(
            p, xs, xd, ea, bond3, upT, qe, skip, last)
        agg2 = _sc_scatter(e_new2, dst, zeros, n_nodes)
        x_in = x_p if skip else xv
        v_out, uv_sumT, ncntT, m_v_k = _node_post(
            p, agg2, degc, x_p, x_in, batch3, upT, qv, last)
        u_inT = upT if skip else stateT
        stateT = _phi_u(p, ue_sumT, bcntT, uv_sumT, ncntT, upT, u_inT)
        xv = v_out
        ea = e_out
        if last:
            m_e, m_v = m_e_k, m_v_k

    denvT, rvT = _s2s_pass2(xv, batch3, m_v, qv, NODE_BLK)
    deneT, reT = _s2s_pass2(ea, bond3, m_e, qe, EDGE_BLK)
    out = _head(p_all, rvT, denvT, reT, deneT, qv, qe, stateT)
    return out.reshape(NUM_GRAPHS, 1)


# speedup vs baseline: 4.5416x; 1.0844x over previous
import functools

import jax
import jax.numpy as jnp
from jax import lax
from jax.experimental import pallas as pl
from jax.experimental.pallas import tpu as pltpu
from jax.experimental.pallas import tpu_sc as plsc

NUM_GRAPHS = 512
EDGE_BLK = 4000
NODE_BLK = 4000
_LOG2 = 0.6931471805599453


_LOG2E = 1.4426950408889634


def _ssp(x):
    r = jnp.log2(1.0 + jnp.exp2(jnp.abs(x) * -_LOG2E))
    return jnp.maximum(x, 0.0) + (r - 1.0) * _LOG2


def _dg(x, w):
    return jax.lax.dot_general(x, w, (((1,), (1,)), ((), ())),
                               preferred_element_type=jnp.float32)


def _dgT(a, b):
    return jax.lax.dot_general(a, b, (((0,), (0,)), ((), ())),
                               preferred_element_type=jnp.float32)


def _matT(w, xT, b):
    return jax.lax.dot_general(w, xT, (((1,), (0,)), ((), ())),
                               preferred_element_type=jnp.float32) + b


def _const_spec(shape):
    return pl.BlockSpec(shape, lambda n: tuple(0 for _ in shape))


def _row_spec(blk, feat):
    return pl.BlockSpec((blk, feat), lambda n: (n, 0))


def _idx_spec(blk):
    return pl.BlockSpec((1, 1, blk), lambda n: (n, 0, 0))


_SC_NW = 32
_GCHUNK = 2000


def _sc_gather_body(table_hbm, idx_hbm, out_hbm, idx_v, rows_v, sem,
                    *, b_per_w, n_ch):
    wid = lax.axis_index("s") * 2 + lax.axis_index("c")
    base = wid * b_per_w

    def body(i, carry):
        off = base + i * _GCHUNK
        pltpu.sync_copy(idx_hbm.at[pl.ds(off, _GCHUNK)], idx_v)
        pltpu.async_copy(table_hbm.at[idx_v], rows_v, sem).wait()
        pltpu.sync_copy(rows_v, out_hbm.at[pl.ds(off, _GCHUNK)])
        return carry

    lax.fori_loop(0, n_ch, body, 0)


def _sc_gather(table, idx):
    B = idx.shape[0]
    D = table.shape[1]
    b_per_w = B // _SC_NW
    n_ch = b_per_w // _GCHUNK
    assert b_per_w % _GCHUNK == 0
    mesh = plsc.VectorSubcoreMesh(core_axis_name="c", subcore_axis_name="s")
    fn = functools.partial(_sc_gather_body, b_per_w=b_per_w, n_ch=n_ch)
    return pl.kernel(
        fn, mesh=mesh,
        compiler_params=pltpu.CompilerParams(use_tc_tiling_on_sc=False),
        out_type=jax.ShapeDtypeStruct((B, D), jnp.float32),
        scratch_types=[
            pltpu.VMEM((_GCHUNK,), jnp.int32),
            pltpu.VMEM((_GCHUNK, D), jnp.float32),
            pltpu.SemaphoreType.DMA,
        ],
    )(table, idx)


_SCHUNK = 1000
_ACC_ROWS = 100096
_ZROWS = 6256


def _sc_scatter_body(vals_hbm, idx_hbm, zeros_hbm, out_hbm, acc, idx_v,
                     vals_v, *, n_edges, n_nodes):
    cid = lax.axis_index("c")
    sid = lax.axis_index("s")
    per_tile = n_edges // 16
    n_ch = per_tile // _SCHUNK
    last = n_nodes - 15 * _ZROWS

    pltpu.sync_copy(zeros_hbm, acc.at[pl.ds(sid * _ZROWS, _ZROWS)])
    plsc.subcore_barrier()

    def body(i, carry):
        off = sid * per_tile + i * _SCHUNK
        pltpu.sync_copy(idx_hbm.at[pl.ds(off, _SCHUNK)], idx_v)
        pltpu.sync_copy(vals_hbm.at[cid, pl.ds(off, _SCHUNK)], vals_v)
        pltpu.sync_copy(vals_v, acc.at[idx_v], add=True)
        return carry

    lax.fori_loop(0, n_ch, body, 0)
    plsc.subcore_barrier()

    @pl.when(sid < 15)
    def _():
        pltpu.sync_copy(acc.at[pl.ds(sid * _ZROWS, _ZROWS)],
                        out_hbm.at[cid, pl.ds(sid * _ZROWS, _ZROWS)])

    @pl.when(sid == 15)
    def _():
        pltpu.sync_copy(acc.at[pl.ds(15 * _ZROWS, last)],
                        out_hbm.at[cid, pl.ds(15 * _ZROWS, last)])


def _sc_scatter(vals2, idx, zeros, n_nodes):
    E = idx.shape[0]
    mesh = plsc.VectorSubcoreMesh(core_axis_name="c", subcore_axis_name="s")
    fn = functools.partial(_sc_scatter_body, n_edges=E, n_nodes=n_nodes)
    return pl.kernel(
        fn, mesh=mesh,
        compiler_params=pltpu.CompilerParams(use_tc_tiling_on_sc=False),
        out_type=jax.ShapeDtypeStruct((2, n_nodes, 16), jnp.float32),
        scratch_types=[
            pltpu.VMEM_SHARED((_ACC_ROWS, 16), jnp.float32),
            pltpu.VMEM((_SCHUNK,), jnp.int32),
            pltpu.VMEM((_SCHUNK, 16), jnp.float32),
        ],
    )(vals2, idx, zeros)


def _edge_kernel(xs_ref, xd_ref, ea_ref, bb_ref, upT_ref, qe_ref,
                 w1_ref, b1_ref, w2_ref, b2_ref,
                 v1_ref, c1_ref, v2_ref, c2_ref, v3_ref, c3_ref,
                 enew_ref, eout_ref, ue_ref, bcnt_ref, me_ref,
                 *, skip_is_ep, compute_max):
    first = pl.program_id(0) == 0
    ea = ea_ref[...]
    e_p = _ssp(_dg(ea, w1_ref[...]) + b1_ref[...])
    e_p = _ssp(_dg(e_p, w2_ref[...]) + b2_ref[...])

    blk = ea.shape[0]
    bb = bb_ref[0, 0, :]
    onehot = (bb[:, None] == jax.lax.broadcasted_iota(
        jnp.int32, (blk, NUM_GRAPHS), 1)).astype(jnp.float32)
    ub = jax.lax.dot_general(onehot, upT_ref[...], (((1,), (1,)), ((), ())),
                             preferred_element_type=jnp.float32)

    h = jnp.concatenate([xs_ref[...], xd_ref[...], e_p, ub], axis=1)
    h = _ssp(_dg(h, v1_ref[...]) + c1_ref[...])
    h = _ssp(_dg(h, v2_ref[...]) + c2_ref[...])
    e_new = _ssp(_dg(h, v3_ref[...]) + c3_ref[...])
    enew_ref[0] = e_new[:, :16]
    enew_ref[1] = e_new[:, 16:]
    if skip_is_ep:
        e_out = e_new + e_p
    else:
        e_out = e_new + ea
    eout_ref[...] = e_out

    @pl.when(first)
    def _():
        ue_ref[...] = jnp.zeros_like(ue_ref)
        bcnt_ref[...] = jnp.zeros_like(bcnt_ref)

    ue_ref[...] += _dgT(e_new, onehot)
    bcnt_ref[...] += jnp.sum(onehot, axis=0, keepdims=True)

    if compute_max:
        ee = jax.lax.dot_general(e_out, qe_ref[...], (((1,), (1,)), ((), ())),
                                 preferred_element_type=jnp.float32)
        masked = jnp.where(onehot > 0.5, ee, -jnp.inf)
        m_part = jnp.max(masked, axis=0, keepdims=True)

        @pl.when(first)
        def _():
            me_ref[...] = jnp.full_like(me_ref, -jnp.inf)

        me_ref[...] = jnp.maximum(me_ref[...], m_part)


def _edge_pipeline(p, xs, xd, edge_attr, bond3, upT, qe, skip_is_ep,
                   compute_max):
    E = xs.shape[0]
    n_blk = E // EDGE_BLK
    w1, b1 = p['pre_e'][0]['W'], p['pre_e'][0]['b']
    w2, b2 = p['pre_e'][1]['W'], p['pre_e'][1]['b']
    v1, c1 = p['phi_e'][0]['W'], p['phi_e'][0]['b']
    v2, c2 = p['phi_e'][1]['W'], p['phi_e'][1]['b']
    v3, c3 = p['phi_e'][2]['W'], p['phi_e'][2]['b']
    e_dim = edge_attr.shape[1]

    out = pl.pallas_call(
        functools.partial(_edge_kernel, skip_is_ep=skip_is_ep,
                          compute_max=compute_max),
        grid=(n_blk,),
        in_specs=[
            _row_spec(EDGE_BLK, 32), _row_spec(EDGE_BLK, 32),
            _row_spec(EDGE_BLK, e_dim), _idx_spec(EDGE_BLK),
            _const_spec((32, NUM_GRAPHS)), _const_spec((1, 32)),
            _const_spec(w1.shape), _const_spec((1, 64)),
            _const_spec(w2.shape), _const_spec((1, 32)),
            _const_spec(v1.shape), _const_spec((1, 64)),
            _const_spec(v2.shape), _const_spec((1, 64)),
            _const_spec(v3.shape), _const_spec((1, 32)),
        ],
        out_specs=[
            pl.BlockSpec((2, EDGE_BLK, 16), lambda n: (0, n, 0)),
            _row_spec(EDGE_BLK, 32),
            _const_spec((32, NUM_GRAPHS)), _const_spec((1, NUM_GRAPHS)),
            _const_spec((1, NUM_GRAPHS)),
        ],
        out_shape=[
            jax.ShapeDtypeStruct((2, E, 16), jnp.float32),
            jax.ShapeDtypeStruct((E, 32), jnp.float32),
            jax.ShapeDtypeStruct((32, NUM_GRAPHS), jnp.float32),
            jax.ShapeDtypeStruct((1, NUM_GRAPHS), jnp.float32),
            jax.ShapeDtypeStruct((1, NUM_GRAPHS), jnp.float32),
        ],
    )(xs, xd, edge_attr, bond3, upT, qe,
      w1, b1.reshape(1, -1), w2, b2.reshape(1, -1),
      v1, c1.reshape(1, -1), v2, c2.reshape(1, -1), v3, c3.reshape(1, -1))
    return out


def _node_pre_kernel(xv_ref, emb_ref, w1_ref, b1_ref, w2_ref, b2_ref,
                     xp_ref, *, with_emb):
    if with_emb:
        xi = xv_ref[0, 0, :]
        blk = xi.shape[0]
        oh = (xi[:, None] == jax.lax.broadcasted_iota(
            jnp.int32, (blk, 95), 1)).astype(jnp.float32)
        xv = jax.lax.dot_general(oh, emb_ref[...], (((1,), (0,)), ((), ())),
                                 preferred_element_type=jnp.float32)
    else:
        xv = xv_ref[...]
    h = _ssp(_dg(xv, w1_ref[...]) + b1_ref[...])
    xp_ref[...] = _ssp(_dg(h, w2_ref[...]) + b2_ref[...])


def _node_pre(p, xv_or_ids, emb, with_emb, n_nodes):
    n_blk = n_nodes // NODE_BLK
    w1, b1 = p['pre_v'][0]['W'], p['pre_v'][0]['b']
    w2, b2 = p['pre_v'][1]['W'], p['pre_v'][1]['b']
    if with_emb:
        first_spec = _idx_spec(NODE_BLK)
    else:
        first_spec = _row_spec(NODE_BLK, xv_or_ids.shape[1])
    return pl.pallas_call(
        functools.partial(_node_pre_kernel, with_emb=with_emb),
        grid=(n_blk,),
        in_specs=[first_spec, _const_spec(emb.shape),
                  _const_spec(w1.shape), _const_spec((1, 64)),
                  _const_spec(w2.shape), _const_spec((1, 32))],
        out_specs=_row_spec(NODE_BLK, 32),
        out_shape=jax.ShapeDtypeStruct((n_nodes, 32), jnp.float32),
    )(xv_or_ids, emb, w1, b1.reshape(1, -1), w2, b2.reshape(1, -1))


def _node_post_kernel(aggs_ref, deg_ref, xp_ref, xin_ref, b_ref, upT_ref,
                      qv_ref, w1_ref, b1_ref, w2_ref, b2_ref, w3_ref, b3_ref,
                      vout_ref, uv_ref, ncnt_ref, mv_ref, *, compute_max):
    first = pl.program_id(0) == 0
    agg = jnp.concatenate([aggs_ref[0], aggs_ref[1]], axis=1) / deg_ref[...]
    blk = agg.shape[0]
    bb = b_ref[0, 0, :]
    onehot = (bb[:, None] == jax.lax.broadcasted_iota(
        jnp.int32, (blk, NUM_GRAPHS), 1)).astype(jnp.float32)
    ub = jax.lax.dot_general(onehot, upT_ref[...], (((1,), (1,)), ((), ())),
                             preferred_element_type=jnp.float32)
    h = jnp.concatenate([agg, xp_ref[...], ub], axis=1)
    h = _ssp(_dg(h, w1_ref[...]) + b1_ref[...])
    h = _ssp(_dg(h, w2_ref[...]) + b2_ref[...])
    v_new = _ssp(_dg(h, w3_ref[...]) + b3_ref[...])
    v_out = v_new + xin_ref[...]
    vout_ref[...] = v_out

    @pl.when(first)
    def _():
        uv_ref[...] = jnp.zeros_like(uv_ref)
        ncnt_ref[...] = jnp.zeros_like(ncnt_ref)

    uv_ref[...] += _dgT(v_new, onehot)
    ncnt_ref[...] += jnp.sum(onehot, axis=0, keepdims=True)

    if compute_max:
        ev = jax.lax.dot_general(v_out, qv_ref[...], (((1,), (1,)), ((), ())),
                                 preferred_element_type=jnp.float32)
        masked = jnp.where(onehot > 0.5, ev, -jnp.inf)
        m_part = jnp.max(masked, axis=0, keepdims=True)

        @pl.when(first)
        def _():
            mv_ref[...] = jnp.full_like(mv_ref, -jnp.inf)

        mv_ref[...] = jnp.maximum(mv_ref[...], m_part)


def _node_post(p, agg_sum, degc, x_p, x_in, batch3, upT, qv, compute_max):
    n_nodes = x_p.shape[0]
    n_blk = n_nodes // NODE_BLK
    w1, b1 = p['phi_v'][0]['W'], p['phi_v'][0]['b']
    w2, b2 = p['phi_v'][1]['W'], p['phi_v'][1]['b']
    w3, b3 = p['phi_v'][2]['W'], p['phi_v'][2]['b']
    return pl.pallas_call(
        functools.partial(_node_post_kernel, compute_max=compute_max),
        grid=(n_blk,),
        in_specs=[
            pl.BlockSpec((2, NODE_BLK, 16), lambda n: (0, n, 0)),
            _row_spec(NODE_BLK, 1),
            _row_spec(NODE_BLK, 32), _row_spec(NODE_BLK, 32),
            _idx_spec(NODE_BLK), _const_spec((32, NUM_GRAPHS)),
            _const_spec((1, 32)),
            _const_spec(w1.shape), _const_spec((1, 64)),
            _const_spec(w2.shape), _const_spec((1, 64)),
            _const_spec(w3.shape), _const_spec((1, 32)),
        ],
        out_specs=[
            _row_spec(NODE_BLK, 32),
            _const_spec((32, NUM_GRAPHS)), _const_spec((1, NUM_GRAPHS)),
            _const_spec((1, NUM_GRAPHS)),
        ],
        out_shape=[
            jax.ShapeDtypeStruct((n_nodes, 32), jnp.float32),
            jax.ShapeDtypeStruct((32, NUM_GRAPHS), jnp.float32),
            jax.ShapeDtypeStruct((1, NUM_GRAPHS), jnp.float32),
            jax.ShapeDtypeStruct((1, NUM_GRAPHS), jnp.float32),
        ],
    )(agg_sum, degc, x_p, x_in, batch3, upT, qv,
      w1, b1.reshape(1, -1), w2, b2.reshape(1, -1), w3, b3.reshape(1, -1))


def _pre_u_kernel(sT_ref, w1_ref, b1_ref, w2_ref, b2_ref, o_ref):
    h = _ssp(_matT(w1_ref[...], sT_ref[...], b1_ref[...]))
    o_ref[...] = _ssp(_matT(w2_ref[...], h, b2_ref[...]))


def _pre_u(p, stateT):
    w1, b1 = p['pre_u'][0]['W'], p['pre_u'][0]['b']
    w2, b2 = p['pre_u'][1]['W'], p['pre_u'][1]['b']
    return pl.pallas_call(
        _pre_u_kernel,
        out_shape=jax.ShapeDtypeStruct((32, NUM_GRAPHS), jnp.float32),
    )(stateT, w1, b1.reshape(-1, 1), w2, b2.reshape(-1, 1))


def _phi_u_kernel(ue_ref, bcnt_ref, uv_ref, ncnt_ref, upT_ref, uinT_ref,
                  w1_ref, b1_ref, w2_ref, b2_ref, w3_ref, b3_ref, o_ref):
    u_e = ue_ref[...] / jnp.maximum(bcnt_ref[...], 1.0)
    u_v = uv_ref[...] / jnp.maximum(ncnt_ref[...], 1.0)
    h = jnp.concatenate([u_e, u_v, upT_ref[...]], axis=0)
    h = _ssp(_matT(w1_ref[...], h, b1_ref[...]))
    h = _ssp(_matT(w2_ref[...], h, b2_ref[...]))
    u_new = _ssp(_matT(w3_ref[...], h, b3_ref[...]))
    o_ref[...] = u_new + uinT_ref[...]


def _phi_u(p, ue_sumT, bcntT, uv_sumT, ncntT, upT, uinT):
    w1, b1 = p['phi_u'][0]['W'], p['phi_u'][0]['b']
    w2, b2 = p['phi_u'][1]['W'], p['phi_u'][1]['b']
    w3, b3 = p['phi_u'][2]['W'], p['phi_u'][2]['b']
    return pl.pallas_call(
        _phi_u_kernel,
        out_shape=jax.ShapeDtypeStruct((32, NUM_GRAPHS), jnp.float32),
    )(ue_sumT, bcntT, uv_sumT, ncntT, upT, uinT,
      w1, b1.reshape(-1, 1), w2, b2.reshape(-1, 1), w3, b3.reshape(-1, 1))


def _s2s_kernel(v_ref, b_ref, m_ref, q_ref, den_ref, r_ref):
    first = pl.program_id(0) == 0
    v = v_ref[...]
    blk = v.shape[0]
    bb = b_ref[0, 0, :]
    onehot = (bb[:, None] == jax.lax.broadcasted_iota(
        jnp.int32, (blk, NUM_GRAPHS), 1)).astype(jnp.float32)
    mm = m_ref[...]
    mm = jnp.where(mm > -1e30, mm, 0.0)
    ee = jax.lax.dot_general(v, q_ref[...], (((1,), (1,)), ((), ())),
                             preferred_element_type=jnp.float32)
    mrow = jax.lax.dot_general(onehot, mm, (((1,), (1,)), ((), ())),
                               preferred_element_type=jnp.float32)
    a = jnp.exp(ee - mrow)

    @pl.when(first)
    def _():
        den_ref[...] = jnp.zeros_like(den_ref)
        r_ref[...] = jnp.zeros_like(r_ref)

    den_ref[...] += jax.lax.dot_general(
        a, onehot, (((0,), (0,)), ((), ())),
        preferred_element_type=jnp.float32)
    r_ref[...] += _dgT(a * v, onehot)


def _s2s_pass2(v, batch3, m, q, blk):
    n = v.shape[0]
    n_blk = n // blk
    return pl.pallas_call(
        _s2s_kernel,
        grid=(n_blk,),
        in_specs=[_row_spec(blk, 32), _idx_spec(blk),
                  _const_spec((1, NUM_GRAPHS)), _const_spec((1, 32))],
        out_specs=[_const_spec((1, NUM_GRAPHS)),
                   _const_spec((32, NUM_GRAPHS))],
        out_shape=[jax.ShapeDtypeStruct((1, NUM_GRAPHS), jnp.float32),
                   jax.ShapeDtypeStruct((32, NUM_GRAPHS), jnp.float32)],
    )(v, batch3, m, q)


def _head_kernel(rv_ref, dv_ref, re_ref, de_ref, qv_ref, qe_ref, sT_ref,
                 w0_ref, b0_ref, w1_ref, b1_ref, w2_ref, b2_ref, o_ref):
    rv = rv_ref[...] / (dv_ref[...] + 1e-16)
    re = re_ref[...] / (de_ref[...] + 1e-16)
    g = rv.shape[1]
    qv = jnp.broadcast_to(qv_ref[...].reshape(32, 1), (32, g))
    qe = jnp.broadcast_to(qe_ref[...].reshape(32, 1), (32, g))
    tmp = jnp.concatenate([qv, rv, qe, re, sT_ref[...]], axis=0)
    h = _ssp(_matT(w0_ref[...], tmp, b0_ref[...]))
    h = _ssp(_matT(w1_ref[...], h, b1_ref[...]))
    o_ref[...] = _matT(w2_ref[...], h, b2_ref[...])


def _head(params, rvT, denvT, reT, deneT, qv, qe, stateT):
    return pl.pallas_call(
        _head_kernel,
        out_shape=jax.ShapeDtypeStruct((1, NUM_GRAPHS), jnp.float32),
    )(rvT, denvT, reT, deneT, qv, qe, stateT,
      params['h0']['W'], params['h0']['b'].reshape(-1, 1),
      params['h1']['W'], params['h1']['b'].reshape(-1, 1),
      params['h2']['W'], params['h2']['b'].reshape(-1, 1))


def _s2s_query(p):
    gates = p['b_ih'] + p['b_hh']
    i, f, g, o = jnp.split(gates, 4)
    c = jax.nn.sigmoid(i) * jnp.tanh(g)
    q = jax.nn.sigmoid(o) * jnp.tanh(c)
    return q.reshape(1, 32)


def kernel(x, edge_index, edge_attr, state, batch, bond_batch, params):
    p_all = params
    src = edge_index[0].astype(jnp.int32)
    dst = edge_index[1].astype(jnp.int32)
    n_nodes = x.shape[0]
    n_edges = edge_attr.shape[0]

    batch3 = batch.astype(jnp.int32).reshape(n_nodes // NODE_BLK, 1, NODE_BLK)
    bond3 = bond_batch.astype(jnp.int32).reshape(
        n_edges // EDGE_BLK, 1, EDGE_BLK)
    x3 = x.astype(jnp.int32).reshape(n_nodes // NODE_BLK, 1, NODE_BLK)

    ones = jnp.ones((n_edges,), jnp.float32)
    deg = jax.ops.segment_sum(ones, dst, num_segments=n_nodes)
    degc = jnp.maximum(deg, 1.0).reshape(n_nodes, 1)
    zeros = jnp.zeros((_ZROWS, 16), jnp.float32)

    stateT = state.T
    qv = _s2s_query(p_all['sv'])
    qe = _s2s_query(p_all['se'])
    dummy_emb = p_all['emb']

    xv = None
    ea = edge_attr
    m_e = m_v = None

    for mi, (pname, skip) in enumerate(
            [('m1', True), ('m2', False), ('m3', False)]):
        p = p_all[pname]
        last = mi == 2
        upT = _pre_u(p, stateT)
        if mi == 0:
            x_p = _node_pre(p, x3, dummy_emb, True, n_nodes)
        else:
            x_p = _node_pre(p, xv, dummy_emb, False, n_nodes)
        xs = _sc_gather(x_p, src)
        xd = _sc_gather(x_p, dst)
        e_new2, e_out, ue_sumT, bcntT, m_e_k = _edge_pipeline(
            p, xs, xd, ea, bond3, upT, qe, skip, last)
        agg2 = _sc_scatter(e_new2, dst, zeros, n_nodes)
        x_in = x_p if skip else xv
        v_out, uv_sumT, ncntT, m_v_k = _node_post(
            p, agg2, degc, x_p, x_in, batch3, upT, qv, last)
        u_inT = upT if skip else stateT
        stateT = _phi_u(p, ue_sumT, bcntT, uv_sumT, ncntT, upT, u_inT)
        xv = v_out
        ea = e_out
        if last:
            m_e, m_v = m_e_k, m_v_k

    denvT, rvT = _s2s_pass2(xv, batch3, m_v, qv, NODE_BLK)
    deneT, reT = _s2s_pass2(ea, bond3, m_e, qe, EDGE_BLK)
    out = _head(p_all, rvT, denvT, reT, deneT, qv, qe, stateT)
    return out.reshape(NUM_GRAPHS, 1)


# speedup vs baseline: 4.5816x; 1.0088x over previous
import functools

import jax
import jax.numpy as jnp
from jax import lax
from jax.experimental import pallas as pl
from jax.experimental.pallas import tpu as pltpu
from jax.experimental.pallas import tpu_sc as plsc

NUM_GRAPHS = 512
EDGE_BLK = 4000
NODE_BLK = 4000
_LOG2 = 0.6931471805599453


_LOG2E = 1.4426950408889634


def _ssp(x):
    r = jnp.log2(1.0 + jnp.exp2(jnp.abs(x) * -_LOG2E))
    return jnp.maximum(x, 0.0) + (r - 1.0) * _LOG2


def _dg(x, w):
    return jax.lax.dot_general(x, w, (((1,), (1,)), ((), ())),
                               preferred_element_type=jnp.float32)


def _dgT(a, b):
    return jax.lax.dot_general(a, b, (((0,), (0,)), ((), ())),
                               preferred_element_type=jnp.float32)


def _matT(w, xT, b):
    return jax.lax.dot_general(w, xT, (((1,), (0,)), ((), ())),
                               preferred_element_type=jnp.float32) + b


def _const_spec(shape):
    return pl.BlockSpec(shape, lambda n: tuple(0 for _ in shape))


def _row_spec(blk, feat):
    return pl.BlockSpec((blk, feat), lambda n: (n, 0))


def _idx_spec(blk):
    return pl.BlockSpec((1, 1, blk), lambda n: (n, 0, 0))


_SC_NW = 32
_GCHUNK = 2000


def _sc_gather_body(table_hbm, idx_hbm, out_hbm, idx_v, rows_v, sem,
                    *, b_per_w, n_ch):
    wid = lax.axis_index("s") * 2 + lax.axis_index("c")
    base = wid * b_per_w

    def body(i, carry):
        off = base + i * _GCHUNK
        pltpu.sync_copy(idx_hbm.at[pl.ds(off, _GCHUNK)], idx_v)
        pltpu.async_copy(table_hbm.at[idx_v], rows_v, sem).wait()
        pltpu.sync_copy(rows_v, out_hbm.at[pl.ds(off, _GCHUNK)])
        return carry

    lax.fori_loop(0, n_ch, body, 0)


def _sc_gather(table, idx):
    B = idx.shape[0]
    D = table.shape[1]
    b_per_w = B // _SC_NW
    n_ch = b_per_w // _GCHUNK
    assert b_per_w % _GCHUNK == 0
    mesh = plsc.VectorSubcoreMesh(core_axis_name="c", subcore_axis_name="s")
    fn = functools.partial(_sc_gather_body, b_per_w=b_per_w, n_ch=n_ch)
    return pl.kernel(
        fn, mesh=mesh,
        compiler_params=pltpu.CompilerParams(use_tc_tiling_on_sc=False),
        out_type=jax.ShapeDtypeStruct((B, D), jnp.float32),
        scratch_types=[
            pltpu.VMEM((_GCHUNK,), jnp.int32),
            pltpu.VMEM((_GCHUNK, D), jnp.float32),
            pltpu.SemaphoreType.DMA,
        ],
    )(table, idx)


_SCHUNK = 1000
_ACC_ROWS = 100096
_ZROWS = 6256


def _sc_scatter_body(vals_hbm, idx_hbm, zeros_hbm, out_hbm, acc, idx_v,
                     vals_v, *, n_edges, n_nodes):
    cid = lax.axis_index("c")
    sid = lax.axis_index("s")
    per_tile = n_edges // 16
    n_ch = per_tile // _SCHUNK
    last = n_nodes - 15 * _ZROWS

    pltpu.sync_copy(zeros_hbm, acc.at[pl.ds(sid * _ZROWS, _ZROWS)])
    plsc.subcore_barrier()

    def body(i, carry):
        off = sid * per_tile + i * _SCHUNK
        pltpu.sync_copy(idx_hbm.at[pl.ds(off, _SCHUNK)], idx_v)
        pltpu.sync_copy(vals_hbm.at[cid, pl.ds(off, _SCHUNK)], vals_v)
        pltpu.sync_copy(vals_v, acc.at[idx_v], add=True)
        return carry

    lax.fori_loop(0, n_ch, body, 0)
    plsc.subcore_barrier()

    @pl.when(sid < 15)
    def _():
        pltpu.sync_copy(acc.at[pl.ds(sid * _ZROWS, _ZROWS)],
                        out_hbm.at[cid, pl.ds(sid * _ZROWS, _ZROWS)])

    @pl.when(sid == 15)
    def _():
        pltpu.sync_copy(acc.at[pl.ds(15 * _ZROWS, last)],
                        out_hbm.at[cid, pl.ds(15 * _ZROWS, last)])


def _sc_scatter(vals2, idx, zeros, n_nodes):
    E = idx.shape[0]
    mesh = plsc.VectorSubcoreMesh(core_axis_name="c", subcore_axis_name="s")
    fn = functools.partial(_sc_scatter_body, n_edges=E, n_nodes=n_nodes)
    return pl.kernel(
        fn, mesh=mesh,
        compiler_params=pltpu.CompilerParams(use_tc_tiling_on_sc=False),
        out_type=jax.ShapeDtypeStruct((2, n_nodes, 16), jnp.float32),
        scratch_types=[
            pltpu.VMEM_SHARED((_ACC_ROWS, 16), jnp.float32),
            pltpu.VMEM((_SCHUNK,), jnp.int32),
            pltpu.VMEM((_SCHUNK, 16), jnp.float32),
        ],
    )(vals2, idx, zeros)


def _edge_kernel(xs_ref, xd_ref, ea_ref, bb_ref, upT_ref, qe_ref,
                 w1_ref, b1_ref, w2_ref, b2_ref,
                 v1_ref, c1_ref, v2_ref, c2_ref, v3_ref, c3_ref,
                 enew_ref, eout_ref, ue_ref, bcnt_ref, me_ref,
                 *, skip_is_ep, compute_max, compute_cnt):
    first = pl.program_id(0) == 0
    ea = ea_ref[...]
    e_p = _ssp(_dg(ea, w1_ref[...]) + b1_ref[...])
    e_p = _ssp(_dg(e_p, w2_ref[...]) + b2_ref[...])

    blk = ea.shape[0]
    bb = bb_ref[0, 0, :]
    oh_bool = (bb[:, None] == jax.lax.broadcasted_iota(
        jnp.int32, (blk, NUM_GRAPHS), 1))
    onehot = oh_bool.astype(jnp.float32)
    ub = jax.lax.dot_general(onehot, upT_ref[...], (((1,), (1,)), ((), ())),
                             preferred_element_type=jnp.float32)

    h = jnp.concatenate([xs_ref[...], xd_ref[...], e_p, ub], axis=1)
    h = _ssp(_dg(h, v1_ref[...]) + c1_ref[...])
    h = _ssp(_dg(h, v2_ref[...]) + c2_ref[...])
    e_new = _ssp(_dg(h, v3_ref[...]) + c3_ref[...])
    enew_ref[0] = e_new[:, :16]
    enew_ref[1] = e_new[:, 16:]
    if skip_is_ep:
        e_out = e_new + e_p
    else:
        e_out = e_new + ea
    eout_ref[...] = e_out

    @pl.when(first)
    def _():
        ue_ref[...] = jnp.zeros_like(ue_ref)
        if compute_cnt:
            bcnt_ref[...] = jnp.zeros_like(bcnt_ref)

    ue_ref[...] += _dgT(e_new, onehot)
    if compute_cnt:
        bcnt_ref[...] += jnp.sum(onehot, axis=0, keepdims=True)

    if compute_max:
        ee = jax.lax.dot_general(e_out, qe_ref[...], (((1,), (1,)), ((), ())),
                                 preferred_element_type=jnp.float32)
        masked = jnp.where(oh_bool, ee, -jnp.inf)
        m_part = jnp.max(masked, axis=0, keepdims=True)

        @pl.when(first)
        def _():
            me_ref[...] = jnp.full_like(me_ref, -jnp.inf)

        me_ref[...] = jnp.maximum(me_ref[...], m_part)


def _edge_pipeline(p, both, edge_attr, bond3, upT, qe, skip_is_ep,
                   compute_max, compute_cnt):
    E = edge_attr.shape[0]
    n_blk = E // EDGE_BLK
    w1, b1 = p['pre_e'][0]['W'], p['pre_e'][0]['b']
    w2, b2 = p['pre_e'][1]['W'], p['pre_e'][1]['b']
    v1, c1 = p['phi_e'][0]['W'], p['phi_e'][0]['b']
    v2, c2 = p['phi_e'][1]['W'], p['phi_e'][1]['b']
    v3, c3 = p['phi_e'][2]['W'], p['phi_e'][2]['b']
    e_dim = edge_attr.shape[1]

    out = pl.pallas_call(
        functools.partial(_edge_kernel, skip_is_ep=skip_is_ep,
                          compute_max=compute_max, compute_cnt=compute_cnt),
        grid=(n_blk,),
        in_specs=[
            _row_spec(EDGE_BLK, 32),
            pl.BlockSpec((EDGE_BLK, 32), lambda n: (n + n_blk, 0)),
            _row_spec(EDGE_BLK, e_dim), _idx_spec(EDGE_BLK),
            _const_spec((32, NUM_GRAPHS)), _const_spec((1, 32)),
            _const_spec(w1.shape), _const_spec((1, 64)),
            _const_spec(w2.shape), _const_spec((1, 32)),
            _const_spec(v1.shape), _const_spec((1, 64)),
            _const_spec(v2.shape), _const_spec((1, 64)),
            _const_spec(v3.shape), _const_spec((1, 32)),
        ],
        out_specs=[
            pl.BlockSpec((2, EDGE_BLK, 16), lambda n: (0, n, 0)),
            _row_spec(EDGE_BLK, 32),
            _const_spec((32, NUM_GRAPHS)), _const_spec((1, NUM_GRAPHS)),
            _const_spec((1, NUM_GRAPHS)),
        ],
        out_shape=[
            jax.ShapeDtypeStruct((2, E, 16), jnp.float32),
            jax.ShapeDtypeStruct((E, 32), jnp.float32),
            jax.ShapeDtypeStruct((32, NUM_GRAPHS), jnp.float32),
            jax.ShapeDtypeStruct((1, NUM_GRAPHS), jnp.float32),
            jax.ShapeDtypeStruct((1, NUM_GRAPHS), jnp.float32),
        ],
    )(both, both, edge_attr, bond3, upT, qe,
      w1, b1.reshape(1, -1), w2, b2.reshape(1, -1),
      v1, c1.reshape(1, -1), v2, c2.reshape(1, -1), v3, c3.reshape(1, -1))
    return out


def _node_pre_kernel(xv_ref, emb_ref, w1_ref, b1_ref, w2_ref, b2_ref,
                     xp_ref, *, with_emb):
    if with_emb:
        xi = xv_ref[0, 0, :]
        blk = xi.shape[0]
        oh = (xi[:, None] == jax.lax.broadcasted_iota(
            jnp.int32, (blk, 95), 1)).astype(jnp.float32)
        xv = jax.lax.dot_general(oh, emb_ref[...], (((1,), (0,)), ((), ())),
                                 preferred_element_type=jnp.float32)
    else:
        xv = xv_ref[...]
    h = _ssp(_dg(xv, w1_ref[...]) + b1_ref[...])
    xp_ref[...] = _ssp(_dg(h, w2_ref[...]) + b2_ref[...])


def _node_pre(p, xv_or_ids, emb, with_emb, n_nodes):
    n_blk = n_nodes // NODE_BLK
    w1, b1 = p['pre_v'][0]['W'], p['pre_v'][0]['b']
    w2, b2 = p['pre_v'][1]['W'], p['pre_v'][1]['b']
    if with_emb:
        first_spec = _idx_spec(NODE_BLK)
    else:
        first_spec = _row_spec(NODE_BLK, xv_or_ids.shape[1])
    return pl.pallas_call(
        functools.partial(_node_pre_kernel, with_emb=with_emb),
        grid=(n_blk,),
        in_specs=[first_spec, _const_spec(emb.shape),
                  _const_spec(w1.shape), _const_spec((1, 64)),
                  _const_spec(w2.shape), _const_spec((1, 32))],
        out_specs=_row_spec(NODE_BLK, 32),
        out_shape=jax.ShapeDtypeStruct((n_nodes, 32), jnp.float32),
    )(xv_or_ids, emb, w1, b1.reshape(1, -1), w2, b2.reshape(1, -1))


def _node_post_kernel(aggs_ref, deg_ref, xp_ref, xin_ref, b_ref, upT_ref,
                      qv_ref, w1_ref, b1_ref, w2_ref, b2_ref, w3_ref, b3_ref,
                      vout_ref, uv_ref, ncnt_ref, mv_ref, *, compute_max):
    first = pl.program_id(0) == 0
    agg = jnp.concatenate([aggs_ref[0], aggs_ref[1]], axis=1) / deg_ref[...]
    blk = agg.shape[0]
    bb = b_ref[0, 0, :]
    onehot = (bb[:, None] == jax.lax.broadcasted_iota(
        jnp.int32, (blk, NUM_GRAPHS), 1)).astype(jnp.float32)
    ub = jax.lax.dot_general(onehot, upT_ref[...], (((1,), (1,)), ((), ())),
                             preferred_element_type=jnp.float32)
    h = jnp.concatenate([agg, xp_ref[...], ub], axis=1)
    h = _ssp(_dg(h, w1_ref[...]) + b1_ref[...])
    h = _ssp(_dg(h, w2_ref[...]) + b2_ref[...])
    v_new = _ssp(_dg(h, w3_ref[...]) + b3_ref[...])
    v_out = v_new + xin_ref[...]
    vout_ref[...] = v_out

    @pl.when(first)
    def _():
        uv_ref[...] = jnp.zeros_like(uv_ref)
        ncnt_ref[...] = jnp.zeros_like(ncnt_ref)

    uv_ref[...] += _dgT(v_new, onehot)
    ncnt_ref[...] += jnp.sum(onehot, axis=0, keepdims=True)

    if compute_max:
        ev = jax.lax.dot_general(v_out, qv_ref[...], (((1,), (1,)), ((), ())),
                                 preferred_element_type=jnp.float32)
        masked = jnp.where(onehot > 0.5, ev, -jnp.inf)
        m_part = jnp.max(masked, axis=0, keepdims=True)

        @pl.when(first)
        def _():
            mv_ref[...] = jnp.full_like(mv_ref, -jnp.inf)

        mv_ref[...] = jnp.maximum(mv_ref[...], m_part)


def _node_post(p, agg_sum, degc, x_p, x_in, batch3, upT, qv, compute_max):
    n_nodes = x_p.shape[0]
    n_blk = n_nodes // NODE_BLK
    w1, b1 = p['phi_v'][0]['W'], p['phi_v'][0]['b']
    w2, b2 = p['phi_v'][1]['W'], p['phi_v'][1]['b']
    w3, b3 = p['phi_v'][2]['W'], p['phi_v'][2]['b']
    return pl.pallas_call(
        functools.partial(_node_post_kernel, compute_max=compute_max),
        grid=(n_blk,),
        in_specs=[
            pl.BlockSpec((2, NODE_BLK, 16), lambda n: (0, n, 0)),
            _row_spec(NODE_BLK, 1),
            _row_spec(NODE_BLK, 32), _row_spec(NODE_BLK, 32),
            _idx_spec(NODE_BLK), _const_spec((32, NUM_GRAPHS)),
            _const_spec((1, 32)),
            _const_spec(w1.shape), _const_spec((1, 64)),
            _const_spec(w2.shape), _const_spec((1, 64)),
            _const_spec(w3.shape), _const_spec((1, 32)),
        ],
        out_specs=[
            _row_spec(NODE_BLK, 32),
            _const_spec((32, NUM_GRAPHS)), _const_spec((1, NUM_GRAPHS)),
            _const_spec((1, NUM_GRAPHS)),
        ],
        out_shape=[
            jax.ShapeDtypeStruct((n_nodes, 32), jnp.float32),
            jax.ShapeDtypeStruct((32, NUM_GRAPHS), jnp.float32),
            jax.ShapeDtypeStruct((1, NUM_GRAPHS), jnp.float32),
            jax.ShapeDtypeStruct((1, NUM_GRAPHS), jnp.float32),
        ],
    )(agg_sum, degc, x_p, x_in, batch3, upT, qv,
      w1, b1.reshape(1, -1), w2, b2.reshape(1, -1), w3, b3.reshape(1, -1))


def _pre_u_kernel(sT_ref, w1_ref, b1_ref, w2_ref, b2_ref, o_ref):
    h = _ssp(_matT(w1_ref[...], sT_ref[...], b1_ref[...]))
    o_ref[...] = _ssp(_matT(w2_ref[...], h, b2_ref[...]))


def _pre_u(p, stateT):
    w1, b1 = p['pre_u'][0]['W'], p['pre_u'][0]['b']
    w2, b2 = p['pre_u'][1]['W'], p['pre_u'][1]['b']
    return pl.pallas_call(
        _pre_u_kernel,
        out_shape=jax.ShapeDtypeStruct((32, NUM_GRAPHS), jnp.float32),
    )(stateT, w1, b1.reshape(-1, 1), w2, b2.reshape(-1, 1))


def _phi_u_kernel(ue_ref, bcnt_ref, uv_ref, ncnt_ref, upT_ref, uinT_ref,
                  w1_ref, b1_ref, w2_ref, b2_ref, w3_ref, b3_ref, o_ref):
    u_e = ue_ref[...] / jnp.maximum(bcnt_ref[...], 1.0)
    u_v = uv_ref[...] / jnp.maximum(ncnt_ref[...], 1.0)
    h = jnp.concatenate([u_e, u_v, upT_ref[...]], axis=0)
    h = _ssp(_matT(w1_ref[...], h, b1_ref[...]))
    h = _ssp(_matT(w2_ref[...], h, b2_ref[...]))
    u_new = _ssp(_matT(w3_ref[...], h, b3_ref[...]))
    o_ref[...] = u_new + uinT_ref[...]


def _phi_u(p, ue_sumT, bcntT, uv_sumT, ncntT, upT, uinT):
    w1, b1 = p['phi_u'][0]['W'], p['phi_u'][0]['b']
    w2, b2 = p['phi_u'][1]['W'], p['phi_u'][1]['b']
    w3, b3 = p['phi_u'][2]['W'], p['phi_u'][2]['b']
    return pl.pallas_call(
        _phi_u_kernel,
        out_shape=jax.ShapeDtypeStruct((32, NUM_GRAPHS), jnp.float32),
    )(ue_sumT, bcntT, uv_sumT, ncntT, upT, uinT,
      w1, b1.reshape(-1, 1), w2, b2.reshape(-1, 1), w3, b3.reshape(-1, 1))


def _s2s_kernel(v_ref, b_ref, m_ref, q_ref, den_ref, r_ref):
    first = pl.program_id(0) == 0
    v = v_ref[...]
    blk = v.shape[0]
    bb = b_ref[0, 0, :]
    onehot = (bb[:, None] == jax.lax.broadcasted_iota(
        jnp.int32, (blk, NUM_GRAPHS), 1)).astype(jnp.float32)
    mm = m_ref[...]
    mm = jnp.where(mm > -1e30, mm, 0.0)
    ee = jax.lax.dot_general(v, q_ref[...], (((1,), (1,)), ((), ())),
                             preferred_element_type=jnp.float32)
    mrow = jax.lax.dot_general(onehot, mm, (((1,), (1,)), ((), ())),
                               preferred_element_type=jnp.float32)
    a = jnp.exp(ee - mrow)

    @pl.when(first)
    def _():
        den_ref[...] = jnp.zeros_like(den_ref)
        r_ref[...] = jnp.zeros_like(r_ref)

    den_ref[...] += jax.lax.dot_general(
        a, onehot, (((0,), (0,)), ((), ())),
        preferred_element_type=jnp.float32)
    r_ref[...] += _dgT(a * v, onehot)


def _s2s_pass2(v, batch3, m, q, blk):
    n = v.shape[0]
    n_blk = n // blk
    return pl.pallas_call(
        _s2s_kernel,
        grid=(n_blk,),
        in_specs=[_row_spec(blk, 32), _idx_spec(blk),
                  _const_spec((1, NUM_GRAPHS)), _const_spec((1, 32))],
        out_specs=[_const_spec((1, NUM_GRAPHS)),
                   _const_spec((32, NUM_GRAPHS))],
        out_shape=[jax.ShapeDtypeStruct((1, NUM_GRAPHS), jnp.float32),
                   jax.ShapeDtypeStruct((32, NUM_GRAPHS), jnp.float32)],
    )(v, batch3, m, q)


def _head_kernel(rv_ref, dv_ref, re_ref, de_ref, qv_ref, qe_ref, sT_ref,
                 w0_ref, b0_ref, w1_ref, b1_ref, w2_ref, b2_ref, o_ref):
    rv = rv_ref[...] / (dv_ref[...] + 1e-16)
    re = re_ref[...] / (de_ref[...] + 1e-16)
    g = rv.shape[1]
    qv = jnp.broadcast_to(qv_ref[...].reshape(32, 1), (32, g))
    qe = jnp.broadcast_to(qe_ref[...].reshape(32, 1), (32, g))
    tmp = jnp.concatenate([qv, rv, qe, re, sT_ref[...]], axis=0)
    h = _ssp(_matT(w0_ref[...], tmp, b0_ref[...]))
    h = _ssp(_matT(w1_ref[...], h, b1_ref[...]))
    o_ref[...] = _matT(w2_ref[...], h, b2_ref[...])


def _head(params, rvT, denvT, reT, deneT, qv, qe, stateT):
    return pl.pallas_call(
        _head_kernel,
        out_shape=jax.ShapeDtypeStruct((1, NUM_GRAPHS), jnp.float32),
    )(rvT, denvT, reT, deneT, qv, qe, stateT,
      params['h0']['W'], params['h0']['b'].reshape(-1, 1),
      params['h1']['W'], params['h1']['b'].reshape(-1, 1),
      params['h2']['W'], params['h2']['b'].reshape(-1, 1))


def _s2s_query(p):
    gates = p['b_ih'] + p['b_hh']
    i, f, g, o = jnp.split(gates, 4)
    c = jax.nn.sigmoid(i) * jnp.tanh(g)
    q = jax.nn.sigmoid(o) * jnp.tanh(c)
    return q.reshape(1, 32)


def kernel(x, edge_index, edge_attr, state, batch, bond_batch, params):
    p_all = params
    src = edge_index[0].astype(jnp.int32)
    dst = edge_index[1].astype(jnp.int32)
    n_nodes = x.shape[0]
    n_edges = edge_attr.shape[0]

    batch3 = batch.astype(jnp.int32).reshape(n_nodes // NODE_BLK, 1, NODE_BLK)
    bond3 = bond_batch.astype(jnp.int32).reshape(
        n_edges // EDGE_BLK, 1, EDGE_BLK)
    x3 = x.astype(jnp.int32).reshape(n_nodes // NODE_BLK, 1, NODE_BLK)

    ones = jnp.ones((n_edges,), jnp.float32)
    deg = jax.ops.segment_sum(ones, dst, num_segments=n_nodes)
    degc = jnp.maximum(deg, 1.0).reshape(n_nodes, 1)
    zeros = jnp.zeros((_ZROWS, 16), jnp.float32)
    srcdst = jnp.concatenate([src, dst])

    stateT = state.T
    qv = _s2s_query(p_all['sv'])
    qe = _s2s_query(p_all['se'])
    dummy_emb = p_all['emb']

    xv = None
    ea = edge_attr
    m_e = m_v = None

    for mi, (pname, skip) in enumerate(
            [('m1', True), ('m2', False), ('m3', False)]):
        p = p_all[pname]
        last = mi == 2
        upT = _pre_u(p, stateT)
        if mi == 0:
            x_p = _node_pre(p, x3, dummy_emb, True, n_nodes)
        else:
            x_p = _node_pre(p, xv, dummy_emb, False, n_nodes)
        both = _sc_gather(x_p, srcdst)
        e_new2, e_out, ue_sumT, bcntT_k, m_e_k = _edge_pipeline(
            p, both, ea, bond3, upT, qe, skip, last, mi == 0)
        if mi == 0:
            bcntT = bcntT_k
        agg2 = _sc_scatter(e_new2, dst, zeros, n_nodes)
        x_in = x_p if skip else xv
        v_out, uv_sumT, ncntT, m_v_k = _node_post(
            p, agg2, degc, x_p, x_in, batch3, upT, qv, last)
        u_inT = upT if skip else stateT
        stateT = _phi_u(p, ue_sumT, bcntT, uv_sumT, ncntT, upT, u_inT)
        xv = v_out
        ea = e_out
        if last:
            m_e, m_v = m_e_k, m_v_k

    denvT, rvT = _s2s_pass2(xv, batch3, m_v, qv, NODE_BLK)
    deneT, reT = _s2s_pass2(ea, bond3, m_e, qe, EDGE_BLK)
    out = _head(p_all, rvT, denvT, reT, deneT, qv, qe, stateT)
    return out.reshape(NUM_GRAPHS, 1)
